# baseline (speedup 1.0000x reference)
import jax
import jax.numpy as jnp
from jax.experimental import pallas as pl
from jax.experimental.pallas import tpu as pltpu


def _seg_sm(scores, seg, n):
    m = jax.ops.segment_max(scores, seg, num_segments=n)
    m = jnp.where(jnp.isfinite(m), m, 0.0)
    e = jnp.exp(scores - m[seg])
    s = jax.ops.segment_sum(e, seg, num_segments=n)
    return e / (s[seg] + 1e-16)


def _gat(x, ei, Wl, bl, Wr, br, att, bias, H, C):
    n = x.shape[0]
    loop = jnp.arange(n, dtype=ei.dtype)
    src = jnp.concatenate([ei[0], loop])
    dst = jnp.concatenate([ei[1], loop])
    xl = (x @ Wl + bl).reshape(n, H, C)
    xr = (x @ Wr + br).reshape(n, H, C)
    m = jax.nn.leaky_relu(xl[src] + xr[dst], 0.2)
    alpha = (m * att[None]).sum(-1)
    alpha = _seg_sm(alpha, dst, n)
    out = jax.ops.segment_sum(xl[src] * alpha[..., None], dst, num_segments=n)
    return out.reshape(n, H * C) + bias


def _final_body(x_ref, w_ref, b_ref, o_ref):
    x = jnp.maximum(x_ref[...], 0.0)
    o_ref[...] = jnp.dot(x, w_ref[...], preferred_element_type=jnp.float32) + b_ref[...]


def _final(x, Wf, bf):
    return pl.pallas_call(
        _final_body,
        out_shape=jax.ShapeDtypeStruct((x.shape[0], Wf.shape[1]), jnp.float32),
    )(x, Wf, bf.reshape(1, -1))


def kernel(x1, edge_index1, x2, edge_index2, x3, edge_index3, Wl1, bl1, Wr1, br1, att1, bias1, Wl2, bl2, Wr2, br2, att2, bias2, Wp, bp, asrc, adst, Wk, bk, q, Wlin, blin, Wf, bf):
    o1 = _gat(x1, edge_index1, Wl1, bl1, Wr1, br1, att1, bias1, 2, 10).reshape(100, 17)
    o2 = _gat(x2, edge_index2, Wl2, bl2, Wr2, br2, att2, bias2, 1, 1).reshape(100, 24)
    n3 = x3.shape[0]
    h = (x3 @ Wp + bp).reshape(n3, 8, 16)
    src = edge_index3[0]
    dst = edge_index3[1]
    a_s = (h * asrc[None]).sum(-1)
    a_d = (h * adst[None]).sum(-1)
    alpha = jax.nn.leaky_relu(a_s[src] + a_d[dst], 0.2)
    alpha = _seg_sm(alpha, dst, n3)
    o = jax.ops.segment_sum(h[src] * alpha[..., None], dst, num_segments=n3).reshape(n3, 128)
    o = jax.nn.relu(o)
    o3 = (o @ Wlin + blin).reshape(100, 17)
    x = jnp.concatenate([o1, o2, o3], axis=1)
    return _final(x, Wf, bf)


# speedup vs baseline: 9.7461x; 9.7461x over previous
import functools

import jax
import jax.numpy as jnp
from jax import lax
from jax.experimental import pallas as pl
from jax.experimental.pallas import tpu as pltpu
from jax.experimental.pallas import tpu_sc as plsc

F32 = jnp.float32
NEG = -3e38

N1, H1, C1 = 85, 2, 10
N2 = 2400
N3, H3, C3 = 85, 8, 16
P1 = 102
E1 = 1536
E2 = 40960
E3 = 1536
NW = 32
E1T, E2T, E3T = E1 // NW, E2 // NW, E3 // NW
G1, G2, G3 = E1T // 16, E2T // 16, E3T // 16

SZ_XL1 = 2048
SZ_XL2 = 2416
SZ_H3 = 10880
SZ_AS = 688
SZ_AD = 816
SZ_D1 = 208
SZ_N1 = 2048
SZ_D2 = 2416
SZ_D3 = 816
SZ_N3 = 13056
SZ_MAX = 192


def _proj_body(x1, w1, b1, x2, w2, b2, x3, wp, bp, a_s, a_d,
               xw1, xw2, h3, as_, ad_):
    h = jnp.dot(x3[...], wp[...], preferred_element_type=F32) + bp[...]
    h3[...] = h
    as_[...] = jnp.dot(h, a_s[...], preferred_element_type=F32)
    ad_[...] = jnp.dot(h, a_d[...], preferred_element_type=F32)
    xw1[...] = jnp.dot(x1[...], w1[...], preferred_element_type=F32) + b1[...]
    xw2[...] = jnp.dot(x2[...], w2[...], preferred_element_type=F32) + b2[...]


def _zero(ref, n):
    def body(i, c):
        ref[pl.ds(i * 16, 16)] = jnp.zeros((16,), F32)
        return c
    lax.fori_loop(0, n // 16, body, 0)


def _sc_body(consts, xl1f, xr1f, xl2f, xr2f, h3f, asf, adf,
             s1h, d1h, s2h, d2h, s3h, d3h,
             maxp, den1p, num1p, den2p, num2p, den3p, num3p,
             constsv, xl1v, xr1v, xl2v, xr2v, h3v, asv, adv,
             s1v, d1v, s2v, d2v, s3v, d3v,
             den1a, num1a, den2a, num2a, den3a, num3a,
             lbuf1, lbuf2, lbuf3, maxs):
    wid = lax.axis_index("s") * 2 + lax.axis_index("c")

    pltpu.sync_copy(consts, constsv)
    pltpu.sync_copy(xl1f, xl1v)
    pltpu.sync_copy(xr1f, xr1v)
    pltpu.sync_copy(xl2f, xl2v)
    pltpu.sync_copy(xr2f, xr2v)
    pltpu.sync_copy(h3f, h3v)
    pltpu.sync_copy(asf, asv)
    pltpu.sync_copy(adf, adv)
    pltpu.sync_copy(s1h.at[pl.ds(wid * E1T, E1T)], s1v)
    pltpu.sync_copy(d1h.at[pl.ds(wid * E1T, E1T)], d1v)
    pltpu.sync_copy(s2h.at[pl.ds(wid * E2T, E2T)], s2v)
    pltpu.sync_copy(d2h.at[pl.ds(wid * E2T, E2T)], d2v)
    pltpu.sync_copy(s3h.at[pl.ds(wid * E3T, E3T)], s3v)
    pltpu.sync_copy(d3h.at[pl.ds(wid * E3T, E3T)], d3v)

    _zero(den1a, SZ_D1)
    _zero(num1a, SZ_N1)
    _zero(den2a, SZ_D2)
    _zero(num2a, SZ_D2)
    _zero(den3a, SZ_D3)
    _zero(num3a, SZ_N3)

    neg = jnp.full((16,), NEG, F32)
    zero16 = jnp.zeros((16,), F32)

    cv0 = constsv[pl.ds(0, 16)]
    cv1 = constsv[pl.ds(16, 16)]

    def att1c(j):
        return cv0[j] if j < 16 else cv1[j - 16]

    att2 = cv1[4]

    def g2p1(i, m):
        s = s2v[pl.ds(i * 16, 16)]
        d = d2v[pl.ds(i * 16, 16)]
        xa = plsc.load_gather(xl2v, [s])
        xb = plsc.load_gather(xr2v, [d])
        t = xa + xb
        l = att2 * jnp.where(t > 0, t, t * 0.2)
        lbuf2[pl.ds(i * 16, 16)] = l
        return jnp.maximum(m, l)

    m2 = lax.fori_loop(0, G2, g2p1, neg)
    maxs[pl.ds(2 * 16, 16)] = m2
    m2s = jnp.max(m2)

    def g2p2(i, c):
        l = lbuf2[pl.ds(i * 16, 16)]
        e = jnp.exp(l - m2s)
        s = s2v[pl.ds(i * 16, 16)]
        d = d2v[pl.ds(i * 16, 16)]
        xa = plsc.load_gather(xl2v, [s])
        plsc.addupdate_scatter(den2a, [d], e)
        plsc.addupdate_scatter(num2a, [d], e * xa)
        return c

    lax.fori_loop(0, G2, g2p2, 0)

    m1 = [neg, neg]
    for i in range(G1):
        s = s1v[pl.ds(i * 16, 16)]
        d = d1v[pl.ds(i * 16, 16)]
        sb = s * 10
        db = d * 10
        for h in range(H1):
            acc = zero16
            for c in range(C1):
                xa = plsc.load_gather(xl1v, [sb + (h * 1020 + c)])
                xb = plsc.load_gather(xr1v, [db + (h * 1020 + c)])
                t = xa + xb
                acc = acc + att1c(h * 10 + c) * jnp.where(t > 0, t, t * 0.2)
            lbuf1[pl.ds(h * E1T + i * 16, 16)] = acc
            m1[h] = jnp.maximum(m1[h], acc)
    m1s = []
    for h in range(H1):
        maxs[pl.ds(h * 16, 16)] = m1[h]
        m1s.append(jnp.max(m1[h]))
    for i in range(G1):
        s = s1v[pl.ds(i * 16, 16)]
        d = d1v[pl.ds(i * 16, 16)]
        sb = s * 10
        db = d * 10
        for h in range(H1):
            l = lbuf1[pl.ds(h * E1T + i * 16, 16)]
            e = jnp.exp(l - m1s[h])
            plsc.addupdate_scatter(den1a, [d + h * 102], e)
            for c in range(C1):
                xa = plsc.load_gather(xl1v, [sb + (h * 1020 + c)])
                plsc.addupdate_scatter(num1a, [db + (h * 1020 + c)], e * xa)

    m3 = [neg] * H3
    for i in range(G3):
        s = s3v[pl.ds(i * 16, 16)]
        d = d3v[pl.ds(i * 16, 16)]
        s8 = s * 8
        d8 = d * 8
        for h in range(H3):
            xa = plsc.load_gather(asv, [s8 + h])
            xb = plsc.load_gather(adv, [d8 + h])
            t = xa + xb
            l = jnp.where(t > 0, t, t * 0.2)
            lbuf3[pl.ds(h * E3T + i * 16, 16)] = l
            m3[h] = jnp.maximum(m3[h], l)
    m3s = []
    for h in range(H3):
        maxs[pl.ds((3 + h) * 16, 16)] = m3[h]
        m3s.append(jnp.max(m3[h]))
    maxs[pl.ds(11 * 16, 16)] = zero16
    for i in range(G3):
        s = s3v[pl.ds(i * 16, 16)]
        d = d3v[pl.ds(i * 16, 16)]
        s128 = s * 128
        d16 = d * 16
        for h in range(H3):
            l = lbuf3[pl.ds(h * E3T + i * 16, 16)]
            e = jnp.exp(l - m3s[h])
            plsc.addupdate_scatter(den3a, [d + h * 102], e)
            for c in range(C3):
                xa = plsc.load_gather(h3v, [s128 + (h * 16 + c)])
                plsc.addupdate_scatter(num3a, [d16 + (h * 1632 + c)], e * xa)

    pltpu.sync_copy(maxs, maxp.at[wid])
    pltpu.sync_copy(den1a, den1p.at[wid])
    pltpu.sync_copy(num1a, num1p.at[wid])
    pltpu.sync_copy(den2a, den2p.at[wid])
    pltpu.sync_copy(num2a, num2p.at[wid])
    pltpu.sync_copy(den3a, den3p.at[wid])
    pltpu.sync_copy(num3a, num3p.at[wid])


_SC_OUT = [
    jax.ShapeDtypeStruct((NW, SZ_MAX), F32),
    jax.ShapeDtypeStruct((NW, SZ_D1), F32),
    jax.ShapeDtypeStruct((NW, SZ_N1), F32),
    jax.ShapeDtypeStruct((NW, SZ_D2), F32),
    jax.ShapeDtypeStruct((NW, SZ_D2), F32),
    jax.ShapeDtypeStruct((NW, SZ_D3), F32),
    jax.ShapeDtypeStruct((NW, SZ_N3), F32),
]

_SC_SCRATCH = [
    pltpu.VMEM((32,), F32),
    pltpu.VMEM((SZ_XL1,), F32),
    pltpu.VMEM((SZ_XL1,), F32),
    pltpu.VMEM((SZ_XL2,), F32),
    pltpu.VMEM((SZ_XL2,), F32),
    pltpu.VMEM((SZ_H3,), F32),
    pltpu.VMEM((SZ_AS,), F32),
    pltpu.VMEM((SZ_AD,), F32),
    pltpu.VMEM((E1T,), jnp.int32),
    pltpu.VMEM((E1T,), jnp.int32),
    pltpu.VMEM((E2T,), jnp.int32),
    pltpu.VMEM((E2T,), jnp.int32),
    pltpu.VMEM((E3T,), jnp.int32),
    pltpu.VMEM((E3T,), jnp.int32),
    pltpu.VMEM((SZ_D1,), F32),
    pltpu.VMEM((SZ_N1,), F32),
    pltpu.VMEM((SZ_D2,), F32),
    pltpu.VMEM((SZ_D2,), F32),
    pltpu.VMEM((SZ_D3,), F32),
    pltpu.VMEM((SZ_N3,), F32),
    pltpu.VMEM((H1 * E1T,), F32),
    pltpu.VMEM((E2T,), F32),
    pltpu.VMEM((H3 * E3T,), F32),
    pltpu.VMEM((SZ_MAX,), F32),
]

@functools.cache
def _sc_kernel():
    return functools.partial(
        pl.kernel,
        out_type=_SC_OUT,
        mesh=plsc.VectorSubcoreMesh(
            core_axis_name="c", subcore_axis_name="s",
            num_cores=2, num_subcores=16),
        scratch_types=_SC_SCRATCH,
        compiler_params=pltpu.CompilerParams(
            use_tc_tiling_on_sc=False, needs_layout_passes=False),
    )(_sc_body)


def _sc_call(consts, xl1f, xr1f, xl2f, xr2f, h3f, asf, adf,
             s1h, d1h, s2h, d2h, s3h, d3h):
    return _sc_kernel()(consts, xl1f, xr1f, xl2f, xr2f, h3f, asf, adf,
                        s1h, d1h, s2h, d2h, s3h, d3h)


def _combine_body(maxp3, d1h0, d1h1, n1h0, n1h1, d2r, n2r,
                  d3h0, d3h1, d3h2, d3h3, d3h4, d3h5, d3h6, d3h7,
                  n3h0, n3h1, n3h2, n3h3, n3h4, n3h5, n3h6, n3h7,
                  wlin, o1o, o2o, o3o):
    mt = jnp.max(maxp3[...], axis=2)
    gm = jnp.max(mt, axis=0)
    sc = jnp.exp(mt - gm[None, :])

    def red3(x, s):
        return jnp.sum(x * s[:, None, None], axis=0)

    def red2(x, s):
        return jnp.sum(x * s[:, None], axis=0)

    o1h = []
    for h, (dh, nh) in enumerate(((d1h0, n1h0), (d1h1, n1h1))):
        den = red2(dh[...], sc[:, h])
        num = red3(nh[...], sc[:, h])
        o1h.append(num / (den[:, None] + 1e-16))
    o1o[...] = jnp.concatenate(o1h, axis=1)

    den2 = red3(d2r[...], sc[:, 2])
    num2 = red3(n2r[...], sc[:, 2])
    o2o[...] = num2 / (den2 + 1e-16)

    w = wlin[...]
    o3 = jnp.zeros((N3, 20), F32)
    d3l = (d3h0, d3h1, d3h2, d3h3, d3h4, d3h5, d3h6, d3h7)
    n3l = (n3h0, n3h1, n3h2, n3h3, n3h4, n3h5, n3h6, n3h7)
    for h in range(H3):
        den = red2(d3l[h][...], sc[:, 3 + h])
        num = red3(n3l[h][...], sc[:, 3 + h])
        pre = jnp.maximum(num / (den[:, None] + 1e-16), 0.0)
        o3 = o3 + jnp.dot(pre, w[h * 16:(h + 1) * 16, :],
                          preferred_element_type=F32)
    o3o[...] = o3


def _final_body(x_ref, b_ref, w_ref, bf_ref, o_ref):
    x = jnp.maximum(x_ref[...] + b_ref[...], 0.0)
    o_ref[...] = jnp.dot(x, w_ref[...], preferred_element_type=F32) + bf_ref[...]


def kernel(x1, edge_index1, x2, edge_index2, x3, edge_index3, Wl1, bl1, Wr1, br1, att1, bias1, Wl2, bl2, Wr2, br2, att2, bias2, Wp, bp, asrc, adst, Wk, bk, q, Wlin, blin, Wf, bf):
    i32 = edge_index1.dtype

    w1 = jnp.concatenate([Wl1, Wr1], axis=1)
    b1 = jnp.concatenate([bl1, br1]).reshape(1, 40)
    w2 = jnp.concatenate([Wl2, Wr2], axis=1)
    b2 = jnp.concatenate([bl2, br2]).reshape(1, 2)
    lanes = jnp.arange(128)
    a_s = jnp.zeros((128, 8), F32).at[lanes, lanes // 16].set(asrc.reshape(128))
    a_d = jnp.zeros((128, 8), F32).at[lanes, lanes // 16].set(adst.reshape(128))

    xw1, xw2, h3, as_, ad_ = pl.pallas_call(
        _proj_body,
        out_shape=[
            jax.ShapeDtypeStruct((N1, 40), F32),
            jax.ShapeDtypeStruct((N2, 2), F32),
            jax.ShapeDtypeStruct((N3, 128), F32),
            jax.ShapeDtypeStruct((N3, 8), F32),
            jax.ShapeDtypeStruct((N3, 8), F32),
        ],
    )(x1, w1, b1, x2, w2, b2, x3, Wp, bp.reshape(1, 128), a_s, a_d)

    def hm1(a):
        a = a.reshape(N1, 2, 10).transpose(1, 0, 2)
        a = jnp.pad(a, ((0, 0), (0, P1 - N1), (0, 0))).reshape(2 * P1 * 10)
        return jnp.pad(a, (0, SZ_XL1 - 2 * P1 * 10))

    xl1f = hm1(xw1[:, :20])
    xr1f = hm1(xw1[:, 20:])
    xl2f = jnp.pad(xw2[:, 0], (0, SZ_XL2 - N2))
    xr2f = jnp.pad(xw2[:, 1], (0, SZ_XL2 - N2))
    h3f = h3.reshape(SZ_H3)
    asf = jnp.pad(as_.reshape(N3 * 8), (0, SZ_AS - N3 * 8))
    adf = jnp.pad(ad_.reshape(N3 * 8), (0, SZ_AD - N3 * 8))
    consts = jnp.pad(jnp.concatenate([att1.reshape(20), att2.reshape(1)]),
                     (0, 11))

    lp1 = jnp.arange(N1, dtype=i32)
    lp2 = jnp.arange(N2, dtype=i32)
    z = lambda k: jnp.zeros((k,), i32)
    dump = lambda n, k: (n + jnp.arange(k, dtype=i32) % 16).astype(i32)
    p1 = E1 - 1360 - N1
    p2 = E2 - 38400 - N2
    p3 = E3 - 1360
    s1h = jnp.concatenate([edge_index1[0], lp1, z(p1)])
    d1h = jnp.concatenate([edge_index1[1], lp1, dump(N1, p1)])
    s2h = jnp.concatenate([edge_index2[0], lp2, z(p2)])
    d2h = jnp.concatenate([edge_index2[1], lp2, dump(N2, p2)])
    s3h = jnp.concatenate([edge_index3[0], z(p3)])
    d3h = jnp.concatenate([edge_index3[1], dump(N3, p3)])

    maxp, den1p, num1p, den2p, num2p, den3p, num3p = _sc_call(
        consts, xl1f, xr1f, xl2f, xr2f, h3f, asf, adf,
        s1h, d1h, s2h, d2h, s3h, d3h)

    maxp3 = maxp.reshape(NW, 12, 16)
    d1 = den1p[:, :2 * P1].reshape(NW, 2, P1)[:, :, :N1]
    n1 = num1p[:, :2 * P1 * 10].reshape(NW, 2, P1, 10)[:, :, :N1, :]
    d2r = den2p[:, :N2].reshape(NW, 100, 24)
    n2r = num2p[:, :N2].reshape(NW, 100, 24)
    d3 = den3p[:, :8 * P1].reshape(NW, 8, P1)[:, :, :N3]
    n3 = num3p.reshape(NW, 8, P1, 16)[:, :, :N3, :]

    o1, o2, o3 = pl.pallas_call(
        _combine_body,
        out_shape=[
            jax.ShapeDtypeStruct((N1, 20), F32),
            jax.ShapeDtypeStruct((100, 24), F32),
            jax.ShapeDtypeStruct((N3, 20), F32),
        ],
    )(maxp3, d1[:, 0], d1[:, 1], n1[:, 0], n1[:, 1], d2r, n2r,
      *[d3[:, h] for h in range(H3)], *[n3[:, h] for h in range(H3)], Wlin)

    xcat = jnp.concatenate(
        [o1.reshape(100, 17), o2, o3.reshape(100, 17)], axis=1)
    bias_cat = jnp.concatenate(
        [jnp.tile(bias1, N1).reshape(100, 17),
         jnp.tile(bias2, N2).reshape(100, 24),
         jnp.tile(blin, N3).reshape(100, 17)], axis=1)

    return pl.pallas_call(
        _final_body,
        out_shape=jax.ShapeDtypeStruct((100, 7), F32),
    )(xcat, bias_cat, Wf, bf.reshape(1, 7))


# speedup vs baseline: 13.2823x; 1.3628x over previous
import functools

import jax
import jax.numpy as jnp
from jax import lax
from jax.experimental import pallas as pl
from jax.experimental.pallas import tpu as pltpu
from jax.experimental.pallas import tpu_sc as plsc

F32 = jnp.float32
NEG = -3e38

N1, H1, C1 = 85, 2, 10
N2 = 2400
N3, H3, C3 = 85, 8, 16
P = 104
E1 = 1536
E2 = 40960
E3 = 1536
NW = 32
E1T, E2T, E3T = E1 // NW, E2 // NW, E3 // NW
G1, G2, G3 = E1T // 16, E2T // 16, E3T // 16

SZ_XW1 = 4160
SZ_XW2 = 4832
SZ_H3 = 10880
SZ_AS = 680
SZ_AD = 832
SZ_D1 = 208
SZ_N1 = 2080
SZ_D2 = 2416
SZ_D3 = 832
SZ_N3 = 13312
SZ_MAX = 192


def _proj_body(x1, w1, b1, x2, w2, b2, x3, wp, bp, a_s, a_d,
               xw1, xw2, h3, as_, ad_):
    h = jnp.dot(x3[...], wp[...], preferred_element_type=F32) + bp[...]
    h3[...] = h
    as_[...] = jnp.dot(h, a_s[...], preferred_element_type=F32)
    ad_[...] = jnp.dot(h, a_d[...], preferred_element_type=F32)
    xw1[...] = jnp.dot(x1[...], w1[...], preferred_element_type=F32) + b1[...]
    xw2[...] = jnp.dot(x2[...], w2[...], preferred_element_type=F32) + b2[...]


def _zero(ref, n):
    z = jnp.zeros((16,), F32)

    def body(i, c):
        base = i * 128
        for j in range(8):
            ref[pl.ds(base + j * 16, 16)] = z
        return c
    lax.fori_loop(0, n // 128, body, 0)
    for k in range(n // 128 * 128, n, 16):
        ref[pl.ds(k, 16)] = z


def _sc_body(consts, xw1f, xw2f, h3f, asf, adf,
             s1h, d1h, s2h, d2h, s3h, d3h,
             maxp, den1p, num1p, den2p, num2p, den3p, num3p,
             constsv, xw1v, xw2v, h3v, asv, adv,
             s1v, d1v, s2v, d2v, s3v, d3v,
             den1a, num1a, den2a, num2a, den3a, num3a,
             lbuf1, lbuf2, lbuf3, maxs, sem):
    wid = lax.axis_index("s") * 2 + lax.axis_index("c")
    z = jnp.zeros((16,), F32)

    copies = [
        pltpu.async_copy(consts, constsv, sem),
        pltpu.async_copy(xw1f, xw1v.at[pl.ds(0, 3400)], sem),
        pltpu.async_copy(xw2f, xw2v.at[pl.ds(0, 4800)], sem),
        pltpu.async_copy(h3f, h3v, sem),
        pltpu.async_copy(asf, asv, sem),
        pltpu.async_copy(adf, adv.at[pl.ds(0, 680)], sem),
        pltpu.async_copy(s1h.at[pl.ds(wid * E1T, E1T)], s1v, sem),
        pltpu.async_copy(d1h.at[pl.ds(wid * E1T, E1T)], d1v, sem),
        pltpu.async_copy(s2h.at[pl.ds(wid * E2T, E2T)], s2v, sem),
        pltpu.async_copy(d2h.at[pl.ds(wid * E2T, E2T)], d2v, sem),
        pltpu.async_copy(s3h.at[pl.ds(wid * E3T, E3T)], s3v, sem),
        pltpu.async_copy(d3h.at[pl.ds(wid * E3T, E3T)], d3v, sem),
    ]

    for k in list(range(3400, 4136, 16)) + [4144]:
        xw1v[pl.ds(k, 16)] = z
    xw2v[pl.ds(4800, 16)] = z
    xw2v[pl.ds(4816, 16)] = z
    for k in list(range(680, 808, 16)) + [816]:
        adv[pl.ds(k, 16)] = z
    _zero(den1a, SZ_D1)
    _zero(num1a, SZ_N1)
    _zero(den2a, SZ_D2)
    _zero(num2a, SZ_D2)
    _zero(den3a, SZ_D3)
    _zero(num3a, SZ_N3)

    for c in copies:
        c.wait()

    neg = jnp.full((16,), NEG, F32)
    cv0 = constsv[pl.ds(0, 16)]
    cv1 = constsv[pl.ds(16, 16)]

    def att1c(j):
        return cv0[j] if j < 16 else cv1[j - 16]

    att2 = cv1[4]

    def g2p1(i, m):
        for u in range(2):
            g = i * 2 + u
            s = s2v[pl.ds(g * 16, 16)]
            d = d2v[pl.ds(g * 16, 16)]
            xa = plsc.load_gather(xw2v, [s * 2])
            xb = plsc.load_gather(xw2v, [d * 2 + 1])
            t = xa + xb
            l = att2 * jnp.where(t > 0, t, t * 0.2)
            lbuf2[pl.ds(g * 16, 16)] = l
            m = jnp.maximum(m, l)
        return m

    m2 = lax.fori_loop(0, G2 // 2, g2p1, neg)
    maxs[pl.ds(2 * 16, 16)] = m2
    m2s = jnp.max(m2)

    def g2p2(i, c):
        for u in range(2):
            g = i * 2 + u
            l = lbuf2[pl.ds(g * 16, 16)]
            e = jnp.exp(l - m2s)
            s = s2v[pl.ds(g * 16, 16)]
            d = d2v[pl.ds(g * 16, 16)]
            xa = plsc.load_gather(xw2v, [s * 2])
            plsc.addupdate_scatter(den2a, [d], e)
            plsc.addupdate_scatter(num2a, [d], e * xa)
        return c

    lax.fori_loop(0, G2 // 2, g2p2, 0)

    m1 = [neg, neg]
    for i in range(G1):
        s = s1v[pl.ds(i * 16, 16)]
        d = d1v[pl.ds(i * 16, 16)]
        sb = s * 40
        db = d * 40
        for h in range(H1):
            acc = z
            for c in range(C1):
                xa = plsc.load_gather(xw1v, [sb + (h * 10 + c)])
                xb = plsc.load_gather(xw1v, [db + (20 + h * 10 + c)])
                t = xa + xb
                acc = acc + att1c(h * 10 + c) * jnp.where(t > 0, t, t * 0.2)
            lbuf1[pl.ds(h * E1T + i * 16, 16)] = acc
            m1[h] = jnp.maximum(m1[h], acc)
    m1s = []
    for h in range(H1):
        maxs[pl.ds(h * 16, 16)] = m1[h]
        m1s.append(jnp.max(m1[h]))
    for i in range(G1):
        s = s1v[pl.ds(i * 16, 16)]
        d = d1v[pl.ds(i * 16, 16)]
        sb = s * 40
        dn = d * 20
        dd = d * 2
        for h in range(H1):
            l = lbuf1[pl.ds(h * E1T + i * 16, 16)]
            e = jnp.exp(l - m1s[h])
            plsc.addupdate_scatter(den1a, [dd + h], e)
            for c in range(C1):
                xa = plsc.load_gather(xw1v, [sb + (h * 10 + c)])
                plsc.addupdate_scatter(num1a, [dn + (h * 10 + c)], e * xa)

    m3 = [neg] * H3
    for i in range(G3):
        s = s3v[pl.ds(i * 16, 16)]
        d = d3v[pl.ds(i * 16, 16)]
        s8 = s * 8
        d8 = d * 8
        for h in range(H3):
            xa = plsc.load_gather(asv, [s8 + h])
            xb = plsc.load_gather(adv, [d8 + h])
            t = xa + xb
            l = jnp.where(t > 0, t, t * 0.2)
            lbuf3[pl.ds(h * E3T + i * 16, 16)] = l
            m3[h] = jnp.maximum(m3[h], l)
    m3s = []
    for h in range(H3):
        maxs[pl.ds((3 + h) * 16, 16)] = m3[h]
        m3s.append(jnp.max(m3[h]))
    maxs[pl.ds(11 * 16, 16)] = z
    for i in range(G3):
        s = s3v[pl.ds(i * 16, 16)]
        d = d3v[pl.ds(i * 16, 16)]
        s128 = s * 128
        d128 = d * 128
        d8 = d * 8
        for h in range(H3):
            l = lbuf3[pl.ds(h * E3T + i * 16, 16)]
            e = jnp.exp(l - m3s[h])
            plsc.addupdate_scatter(den3a, [d8 + h], e)
            for c in range(C3):
                xa = plsc.load_gather(h3v, [s128 + (h * 16 + c)])
                plsc.addupdate_scatter(num3a, [d128 + (h * 16 + c)], e * xa)

    pltpu.sync_copy(maxs, maxp.at[wid])
    pltpu.sync_copy(den1a, den1p.at[wid])
    pltpu.sync_copy(num1a, num1p.at[wid])
    pltpu.sync_copy(den2a, den2p.at[wid])
    pltpu.sync_copy(num2a, num2p.at[wid])
    pltpu.sync_copy(den3a, den3p.at[wid])
    pltpu.sync_copy(num3a, num3p.at[wid])


_SC_OUT = [
    jax.ShapeDtypeStruct((NW, SZ_MAX), F32),
    jax.ShapeDtypeStruct((NW, SZ_D1), F32),
    jax.ShapeDtypeStruct((NW, SZ_N1), F32),
    jax.ShapeDtypeStruct((NW, SZ_D2), F32),
    jax.ShapeDtypeStruct((NW, SZ_D2), F32),
    jax.ShapeDtypeStruct((NW, SZ_D3), F32),
    jax.ShapeDtypeStruct((NW, SZ_N3), F32),
]

_SC_SCRATCH = [
    pltpu.VMEM((32,), F32),
    pltpu.VMEM((SZ_XW1,), F32),
    pltpu.VMEM((SZ_XW2,), F32),
    pltpu.VMEM((SZ_H3,), F32),
    pltpu.VMEM((SZ_AS,), F32),
    pltpu.VMEM((SZ_AD,), F32),
    pltpu.VMEM((E1T,), jnp.int32),
    pltpu.VMEM((E1T,), jnp.int32),
    pltpu.VMEM((E2T,), jnp.int32),
    pltpu.VMEM((E2T,), jnp.int32),
    pltpu.VMEM((E3T,), jnp.int32),
    pltpu.VMEM((E3T,), jnp.int32),
    pltpu.VMEM((SZ_D1,), F32),
    pltpu.VMEM((SZ_N1,), F32),
    pltpu.VMEM((SZ_D2,), F32),
    pltpu.VMEM((SZ_D2,), F32),
    pltpu.VMEM((SZ_D3,), F32),
    pltpu.VMEM((SZ_N3,), F32),
    pltpu.VMEM((H1 * E1T,), F32),
    pltpu.VMEM((E2T,), F32),
    pltpu.VMEM((H3 * E3T,), F32),
    pltpu.VMEM((SZ_MAX,), F32),
    pltpu.SemaphoreType.DMA,
]


@functools.cache
def _sc_kernel():
    return functools.partial(
        pl.kernel,
        out_type=_SC_OUT,
        mesh=plsc.VectorSubcoreMesh(
            core_axis_name="c", subcore_axis_name="s",
            num_cores=2, num_subcores=16),
        scratch_types=_SC_SCRATCH,
        compiler_params=pltpu.CompilerParams(
            use_tc_tiling_on_sc=False, needs_layout_passes=False),
    )(_sc_body)


def _sc_call(*args):
    return _sc_kernel()(*args)


def _combine_body(maxp3, d1p, n1p, d2p, n2p, d3p, n3p,
                  e1, e3, wlin, b1, b2, blin, t1o, t2o, t3o):
    mt = jnp.max(maxp3[...], axis=2)
    gm = jnp.max(mt, axis=0)
    sc = jnp.exp(mt - gm[None, :])
    e1m = e1[...]
    e3m = e3[...]

    sc1 = sc[:, 0:2]
    den1 = jnp.sum(d1p[...] * sc1[:, None, :], axis=0)
    s1pat = jnp.dot(sc1, e1m, preferred_element_type=F32)
    num1 = jnp.sum(n1p[...] * s1pat[:, None, :], axis=0)
    dex1 = jnp.dot(den1, e1m, preferred_element_type=F32)
    t1o[...] = jnp.maximum(num1 / (dex1 + 1e-16) + b1[...], 0.0)

    sc2 = sc[:, 2:3]
    den2 = jnp.sum(d2p[...] * sc2, axis=0, keepdims=True)
    num2 = jnp.sum(n2p[...] * sc2, axis=0, keepdims=True)
    t2o[...] = jnp.maximum(num2 / (den2 + 1e-16) + b2[...], 0.0)

    sc3 = sc[:, 3:11]
    den3 = jnp.sum(d3p[...] * sc3[:, None, :], axis=0)
    s3pat = jnp.dot(sc3, e3m, preferred_element_type=F32)
    num3 = jnp.sum(n3p[...] * s3pat[:, None, :], axis=0)
    dex3 = jnp.dot(den3, e3m, preferred_element_type=F32)
    pre = jnp.maximum(num3 / (dex3 + 1e-16), 0.0)
    o3 = jnp.dot(pre, wlin[...], preferred_element_type=F32) + blin[...]
    t3o[...] = jnp.maximum(o3, 0.0)


def _final_body(x_ref, w_ref, bf_ref, o_ref):
    o_ref[...] = jnp.dot(x_ref[...], w_ref[...],
                         preferred_element_type=F32) + bf_ref[...]


def kernel(x1, edge_index1, x2, edge_index2, x3, edge_index3, Wl1, bl1, Wr1, br1, att1, bias1, Wl2, bl2, Wr2, br2, att2, bias2, Wp, bp, asrc, adst, Wk, bk, q, Wlin, blin, Wf, bf):
    i32 = edge_index1.dtype

    w1 = jnp.concatenate([Wl1, Wr1], axis=1)
    b1 = jnp.concatenate([bl1, br1]).reshape(1, 40)
    w2 = jnp.concatenate([Wl2, Wr2], axis=1)
    b2 = jnp.concatenate([bl2, br2]).reshape(1, 2)
    lanes = jnp.arange(128)
    a_s = jnp.zeros((128, 8), F32).at[lanes, lanes // 16].set(asrc.reshape(128))
    a_d = jnp.zeros((128, 8), F32).at[lanes, lanes // 16].set(adst.reshape(128))

    xw1, xw2, h3, as_, ad_ = pl.pallas_call(
        _proj_body,
        out_shape=[
            jax.ShapeDtypeStruct((N1, 40), F32),
            jax.ShapeDtypeStruct((N2, 2), F32),
            jax.ShapeDtypeStruct((N3, 128), F32),
            jax.ShapeDtypeStruct((N3, 8), F32),
            jax.ShapeDtypeStruct((N3, 8), F32),
        ],
    )(x1, w1, b1, x2, w2, b2, x3, Wp, bp.reshape(1, 128), a_s, a_d)

    consts = jnp.pad(jnp.concatenate([att1.reshape(20), att2.reshape(1)]),
                     (0, 11))
    lp1 = jnp.arange(N1, dtype=i32)
    lp2 = jnp.arange(N2, dtype=i32)
    zk = lambda k: jnp.zeros((k,), i32)
    dump = lambda n, k: (n + jnp.arange(k, dtype=i32) % 16).astype(i32)
    p1 = E1 - 1360 - N1
    p2 = E2 - 38400 - N2
    p3 = E3 - 1360
    s1h = jnp.concatenate([edge_index1[0], lp1, zk(p1)])
    d1h = jnp.concatenate([edge_index1[1], lp1, dump(N1, p1)])
    s2h = jnp.concatenate([edge_index2[0], lp2, zk(p2)])
    d2h = jnp.concatenate([edge_index2[1], lp2, dump(N2, p2)])
    s3h = jnp.concatenate([edge_index3[0], zk(p3)])
    d3h = jnp.concatenate([edge_index3[1], dump(N3, p3)])

    maxp, den1p, num1p, den2p, num2p, den3p, num3p = _sc_call(
        consts, xw1.reshape(N1 * 40), xw2.reshape(N2 * 2),
        h3.reshape(SZ_H3), as_.reshape(N3 * 8), ad_.reshape(N3 * 8),
        s1h, d1h, s2h, d2h, s3h, d3h)

    e1m = jnp.repeat(jnp.eye(2, dtype=F32), 10, axis=1)
    e3m = jnp.repeat(jnp.eye(8, dtype=F32), 16, axis=1)
    t1, t2, t3 = pl.pallas_call(
        _combine_body,
        out_shape=[
            jax.ShapeDtypeStruct((P, 20), F32),
            jax.ShapeDtypeStruct((1, SZ_D2), F32),
            jax.ShapeDtypeStruct((P, 20), F32),
        ],
    )(maxp.reshape(NW, 12, 16), den1p.reshape(NW, P, 2),
      num1p.reshape(NW, P, 20), den2p, num2p,
      den3p.reshape(NW, P, 8), num3p.reshape(NW, P, 128),
      e1m, e3m, Wlin, bias1.reshape(1, 20), bias2.reshape(1, 1),
      blin.reshape(1, 20))

    xcat = jnp.concatenate(
        [t1[:N1].reshape(100, 17), t2[0, :N2].reshape(100, 24),
         t3[:N3].reshape(100, 17)], axis=1)
    return pl.pallas_call(
        _final_body,
        out_shape=jax.ShapeDtypeStruct((100, 7), F32),
    )(xcat, Wf, bf.reshape(1, 7))


# speedup vs baseline: 13.6663x; 1.0289x over previous
import functools

import jax
import jax.numpy as jnp
from jax import lax
from jax.experimental import pallas as pl
from jax.experimental.pallas import tpu as pltpu
from jax.experimental.pallas import tpu_sc as plsc

F32 = jnp.float32
NEG = -3e38

N1, H1, C1 = 85, 2, 10
N2 = 2400
N3, H3, C3 = 85, 8, 16
P = 104
E1 = 1536
E2 = 40960
E3 = 1536
NW = 32
E1T, E2T, E3T = E1 // NW, E2 // NW, E3 // NW
G1, G2, G3 = E1T // 16, E2T // 16, E3T // 16

SZ_XW1 = 4160
SZ_XW2 = 4832
SZ_H3 = 10880
SZ_AS = 680
SZ_AD = 832
SZ_D1 = 208
SZ_N1 = 2080
SZ_D2 = 2416
SZ_D3 = 832
SZ_N3 = 13312
SZ_MAX = 192


def _proj_body(x1, wl1, bl1, wr1, br1, x2, wl2, bl2, wr2, br2,
               x3, wp, bp, asrc, adst, att1, att2,
               xw1, xw2, h3, as_, ad_, consts):
    h = jnp.dot(x3[...], wp[...], preferred_element_type=F32) + bp[...]
    h3[...] = h
    asm = asrc[...]
    adm = adst[...]
    acols, dcols = [], []
    for hh in range(8):
        blk = h[:, hh * 16:(hh + 1) * 16]
        acols.append(jnp.sum(blk * asm[hh:hh + 1, :], axis=1, keepdims=True))
        dcols.append(jnp.sum(blk * adm[hh:hh + 1, :], axis=1, keepdims=True))
    as_[...] = jnp.concatenate(acols, axis=1)
    ad_[...] = jnp.concatenate(dcols, axis=1)
    xw1[...] = jnp.concatenate(
        [jnp.dot(x1[...], wl1[...], preferred_element_type=F32) + bl1[...],
         jnp.dot(x1[...], wr1[...], preferred_element_type=F32) + br1[...]],
        axis=1)
    xw2[...] = jnp.concatenate(
        [jnp.dot(x2[...], wl2[...], preferred_element_type=F32) + bl2[...],
         jnp.dot(x2[...], wr2[...], preferred_element_type=F32) + br2[...]],
        axis=1)
    a1 = att1[...]
    consts[...] = jnp.concatenate(
        [a1[0:1, :], a1[1:2, :], att2[...], jnp.zeros((1, 11), F32)], axis=1)


def _zero(ref, n):
    z = jnp.zeros((16,), F32)

    def body(i, c):
        base = i * 128
        for j in range(8):
            ref[pl.ds(base + j * 16, 16)] = z
        return c
    lax.fori_loop(0, n // 128, body, 0)
    for k in range(n // 128 * 128, n, 16):
        ref[pl.ds(k, 16)] = z


OFF_S1 = 0
OFF_D1 = E1
OFF_S2 = 2 * E1
OFF_D2 = 2 * E1 + E2
OFF_S3 = 2 * E1 + 2 * E2
OFF_D3 = 3 * E1 + 2 * E2


def _sc_body(consts, xw1f, xw2f, h3f, asf, adf, edges,
             maxp, den1p, num1p, den2p, num2p, den3p, num3p,
             constsv, xw1v, xw2v, h3v, asv, adv,
             s1v, d1v, s2v, d2v, s3v, d3v,
             den1a, num1a, den2a, num2a, den3a, num3a,
             lbuf1, lbuf2, lbuf3, maxs, sem):
    wid = lax.axis_index("s") * 2 + lax.axis_index("c")
    z = jnp.zeros((16,), F32)

    copies = [
        pltpu.async_copy(consts, constsv, sem),
        pltpu.async_copy(xw1f, xw1v.at[pl.ds(0, 3400)], sem),
        pltpu.async_copy(xw2f, xw2v.at[pl.ds(0, 4800)], sem),
        pltpu.async_copy(h3f, h3v, sem),
        pltpu.async_copy(asf, asv, sem),
        pltpu.async_copy(adf, adv.at[pl.ds(0, 680)], sem),
        pltpu.async_copy(edges.at[pl.ds(OFF_S1 + wid * E1T, E1T)], s1v, sem),
        pltpu.async_copy(edges.at[pl.ds(OFF_D1 + wid * E1T, E1T)], d1v, sem),
        pltpu.async_copy(edges.at[pl.ds(OFF_S2 + wid * E2T, E2T)], s2v, sem),
        pltpu.async_copy(edges.at[pl.ds(OFF_D2 + wid * E2T, E2T)], d2v, sem),
        pltpu.async_copy(edges.at[pl.ds(OFF_S3 + wid * E3T, E3T)], s3v, sem),
        pltpu.async_copy(edges.at[pl.ds(OFF_D3 + wid * E3T, E3T)], d3v, sem),
    ]

    for k in list(range(3400, 4136, 16)) + [4144]:
        xw1v[pl.ds(k, 16)] = z
    xw2v[pl.ds(4800, 16)] = z
    xw2v[pl.ds(4816, 16)] = z
    for k in list(range(680, 808, 16)) + [816]:
        adv[pl.ds(k, 16)] = z
    _zero(den1a, SZ_D1)
    _zero(num1a, SZ_N1)
    _zero(den2a, SZ_D2)
    _zero(num2a, SZ_D2)
    _zero(den3a, SZ_D3)
    _zero(num3a, SZ_N3)

    for c in copies:
        c.wait()

    neg = jnp.full((16,), NEG, F32)
    cv0 = constsv[pl.ds(0, 16)]
    cv1 = constsv[pl.ds(16, 16)]

    def att1c(j):
        return cv0[j] if j < 16 else cv1[j - 16]

    att2 = cv1[4]

    def g2p1(i, m):
        for u in range(2):
            g = i * 2 + u
            s = s2v[pl.ds(g * 16, 16)]
            d = d2v[pl.ds(g * 16, 16)]
            xa = plsc.load_gather(xw2v, [s * 2])
            xb = plsc.load_gather(xw2v, [d * 2 + 1])
            t = xa + xb
            l = att2 * jnp.where(t > 0, t, t * 0.2)
            lbuf2[pl.ds(g * 16, 16)] = l
            m = jnp.maximum(m, l)
        return m

    m2 = lax.fori_loop(0, G2 // 2, g2p1, neg)
    maxs[pl.ds(2 * 16, 16)] = m2
    m2s = jnp.max(m2)

    def g2p2(i, c):
        for u in range(2):
            g = i * 2 + u
            l = lbuf2[pl.ds(g * 16, 16)]
            e = jnp.exp(l - m2s)
            s = s2v[pl.ds(g * 16, 16)]
            d = d2v[pl.ds(g * 16, 16)]
            xa = plsc.load_gather(xw2v, [s * 2])
            plsc.addupdate_scatter(den2a, [d], e)
            plsc.addupdate_scatter(num2a, [d], e * xa)
        return c

    lax.fori_loop(0, G2 // 2, g2p2, 0)

    m1 = [neg, neg]
    for i in range(G1):
        s = s1v[pl.ds(i * 16, 16)]
        d = d1v[pl.ds(i * 16, 16)]
        sb = s * 40
        db = d * 40
        for h in range(H1):
            acc = z
            for c in range(C1):
                xa = plsc.load_gather(xw1v, [sb + (h * 10 + c)])
                xb = plsc.load_gather(xw1v, [db + (20 + h * 10 + c)])
                t = xa + xb
                acc = acc + att1c(h * 10 + c) * jnp.where(t > 0, t, t * 0.2)
            lbuf1[pl.ds(h * E1T + i * 16, 16)] = acc
            m1[h] = jnp.maximum(m1[h], acc)
    m1s = []
    for h in range(H1):
        maxs[pl.ds(h * 16, 16)] = m1[h]
        m1s.append(jnp.max(m1[h]))
    for i in range(G1):
        s = s1v[pl.ds(i * 16, 16)]
        d = d1v[pl.ds(i * 16, 16)]
        sb = s * 40
        dn = d * 20
        dd = d * 2
        for h in range(H1):
            l = lbuf1[pl.ds(h * E1T + i * 16, 16)]
            e = jnp.exp(l - m1s[h])
            plsc.addupdate_scatter(den1a, [dd + h], e)
            for c in range(C1):
                xa = plsc.load_gather(xw1v, [sb + (h * 10 + c)])
                plsc.addupdate_scatter(num1a, [dn + (h * 10 + c)], e * xa)

    m3 = [neg] * H3
    for i in range(G3):
        s = s3v[pl.ds(i * 16, 16)]
        d = d3v[pl.ds(i * 16, 16)]
        s8 = s * 8
        d8 = d * 8
        for h in range(H3):
            xa = plsc.load_gather(asv, [s8 + h])
            xb = plsc.load_gather(adv, [d8 + h])
            t = xa + xb
            l = jnp.where(t > 0, t, t * 0.2)
            lbuf3[pl.ds(h * E3T + i * 16, 16)] = l
            m3[h] = jnp.maximum(m3[h], l)
    m3s = []
    for h in range(H3):
        maxs[pl.ds((3 + h) * 16, 16)] = m3[h]
        m3s.append(jnp.max(m3[h]))
    maxs[pl.ds(11 * 16, 16)] = z
    for i in range(G3):
        s = s3v[pl.ds(i * 16, 16)]
        d = d3v[pl.ds(i * 16, 16)]
        s128 = s * 128
        d128 = d * 128
        d8 = d * 8
        for h in range(H3):
            l = lbuf3[pl.ds(h * E3T + i * 16, 16)]
            e = jnp.exp(l - m3s[h])
            plsc.addupdate_scatter(den3a, [d8 + h], e)
            for c in range(C3):
                xa = plsc.load_gather(h3v, [s128 + (h * 16 + c)])
                plsc.addupdate_scatter(num3a, [d128 + (h * 16 + c)], e * xa)

    pltpu.sync_copy(maxs, maxp.at[wid])
    pltpu.sync_copy(den1a, den1p.at[wid])
    pltpu.sync_copy(num1a, num1p.at[wid])
    pltpu.sync_copy(den2a, den2p.at[wid])
    pltpu.sync_copy(num2a, num2p.at[wid])
    pltpu.sync_copy(den3a, den3p.at[wid])
    pltpu.sync_copy(num3a, num3p.at[wid])


_SC_OUT = [
    jax.ShapeDtypeStruct((NW, SZ_MAX), F32),
    jax.ShapeDtypeStruct((NW, SZ_D1), F32),
    jax.ShapeDtypeStruct((NW, SZ_N1), F32),
    jax.ShapeDtypeStruct((NW, SZ_D2), F32),
    jax.ShapeDtypeStruct((NW, SZ_D2), F32),
    jax.ShapeDtypeStruct((NW, SZ_D3), F32),
    jax.ShapeDtypeStruct((NW, SZ_N3), F32),
]

_SC_SCRATCH = [
    pltpu.VMEM((32,), F32),
    pltpu.VMEM((SZ_XW1,), F32),
    pltpu.VMEM((SZ_XW2,), F32),
    pltpu.VMEM((SZ_H3,), F32),
    pltpu.VMEM((SZ_AS,), F32),
    pltpu.VMEM((SZ_AD,), F32),
    pltpu.VMEM((E1T,), jnp.int32),
    pltpu.VMEM((E1T,), jnp.int32),
    pltpu.VMEM((E2T,), jnp.int32),
    pltpu.VMEM((E2T,), jnp.int32),
    pltpu.VMEM((E3T,), jnp.int32),
    pltpu.VMEM((E3T,), jnp.int32),
    pltpu.VMEM((SZ_D1,), F32),
    pltpu.VMEM((SZ_N1,), F32),
    pltpu.VMEM((SZ_D2,), F32),
    pltpu.VMEM((SZ_D2,), F32),
    pltpu.VMEM((SZ_D3,), F32),
    pltpu.VMEM((SZ_N3,), F32),
    pltpu.VMEM((H1 * E1T,), F32),
    pltpu.VMEM((E2T,), F32),
    pltpu.VMEM((H3 * E3T,), F32),
    pltpu.VMEM((SZ_MAX,), F32),
    pltpu.SemaphoreType.DMA,
]


@functools.cache
def _sc_kernel():
    return functools.partial(
        pl.kernel,
        out_type=_SC_OUT,
        mesh=plsc.VectorSubcoreMesh(
            core_axis_name="c", subcore_axis_name="s",
            num_cores=2, num_subcores=16),
        scratch_types=_SC_SCRATCH,
        compiler_params=pltpu.CompilerParams(
            use_tc_tiling_on_sc=False, needs_layout_passes=False),
    )(_sc_body)


def _sc_call(*args):
    return _sc_kernel()(*args)


def _combine_body(maxp3, d1p, n1p, d2p, n2p, d3p, n3p,
                  e1, e3, wlin, b1, b2, blin, t1o, t2o, t3o):
    mt = jnp.max(maxp3[...], axis=2)
    gm = jnp.max(mt, axis=0)
    sc = jnp.exp(mt - gm[None, :])
    e1m = e1[...]
    e3m = e3[...]

    sc1 = sc[:, 0:2]
    den1 = jnp.sum(d1p[...] * sc1[:, None, :], axis=0)
    s1pat = jnp.dot(sc1, e1m, preferred_element_type=F32)
    num1 = jnp.sum(n1p[...] * s1pat[:, None, :], axis=0)
    dex1 = jnp.dot(den1, e1m, preferred_element_type=F32)
    t1o[...] = jnp.maximum(num1 / (dex1 + 1e-16) + b1[...], 0.0)

    sc2 = sc[:, 2:3]
    den2 = jnp.sum(d2p[...] * sc2, axis=0, keepdims=True)
    num2 = jnp.sum(n2p[...] * sc2, axis=0, keepdims=True)
    t2o[...] = jnp.maximum(num2 / (den2 + 1e-16) + b2[...], 0.0)

    sc3 = sc[:, 3:11]
    den3 = jnp.sum(d3p[...] * sc3[:, None, :], axis=0)
    s3pat = jnp.dot(sc3, e3m, preferred_element_type=F32)
    num3 = jnp.sum(n3p[...] * s3pat[:, None, :], axis=0)
    dex3 = jnp.dot(den3, e3m, preferred_element_type=F32)
    pre = jnp.maximum(num3 / (dex3 + 1e-16), 0.0)
    o3 = jnp.dot(pre, wlin[...], preferred_element_type=F32) + blin[...]
    t3o[...] = jnp.maximum(o3, 0.0)


def _final_body(x_ref, w_ref, bf_ref, o_ref):
    o_ref[...] = jnp.dot(x_ref[...], w_ref[...],
                         preferred_element_type=F32) + bf_ref[...]


def kernel(x1, edge_index1, x2, edge_index2, x3, edge_index3, Wl1, bl1, Wr1, br1, att1, bias1, Wl2, bl2, Wr2, br2, att2, bias2, Wp, bp, asrc, adst, Wk, bk, q, Wlin, blin, Wf, bf):
    i32 = edge_index1.dtype

    xw1, xw2, h3, as_, ad_, consts = pl.pallas_call(
        _proj_body,
        out_shape=[
            jax.ShapeDtypeStruct((N1, 40), F32),
            jax.ShapeDtypeStruct((N2, 2), F32),
            jax.ShapeDtypeStruct((N3, 128), F32),
            jax.ShapeDtypeStruct((N3, 8), F32),
            jax.ShapeDtypeStruct((N3, 8), F32),
            jax.ShapeDtypeStruct((1, 32), F32),
        ],
    )(x1, Wl1, bl1.reshape(1, 20), Wr1, br1.reshape(1, 20),
      x2, Wl2, bl2.reshape(1, 1), Wr2, br2.reshape(1, 1),
      x3, Wp, bp.reshape(1, 128), asrc, adst, att1, att2)

    lp1 = jnp.arange(N1, dtype=i32)
    lp2 = jnp.arange(N2, dtype=i32)
    zk = lambda k: jnp.zeros((k,), i32)
    dump = lambda n, k: (n + jnp.arange(k, dtype=i32) % 16).astype(i32)
    p1 = E1 - 1360 - N1
    p2 = E2 - 38400 - N2
    p3 = E3 - 1360
    edges = jnp.concatenate([
        edge_index1[0], lp1, zk(p1),
        edge_index1[1], lp1, dump(N1, p1),
        edge_index2[0], lp2, zk(p2),
        edge_index2[1], lp2, dump(N2, p2),
        edge_index3[0], zk(p3),
        edge_index3[1], dump(N3, p3),
    ])

    maxp, den1p, num1p, den2p, num2p, den3p, num3p = _sc_call(
        consts.reshape(32), xw1.reshape(N1 * 40), xw2.reshape(N2 * 2),
        h3.reshape(SZ_H3), as_.reshape(N3 * 8), ad_.reshape(N3 * 8), edges)

    e1m = jnp.repeat(jnp.eye(2, dtype=F32), 10, axis=1)
    e3m = jnp.repeat(jnp.eye(8, dtype=F32), 16, axis=1)
    t1, t2, t3 = pl.pallas_call(
        _combine_body,
        out_shape=[
            jax.ShapeDtypeStruct((P, 20), F32),
            jax.ShapeDtypeStruct((1, SZ_D2), F32),
            jax.ShapeDtypeStruct((P, 20), F32),
        ],
    )(maxp.reshape(NW, 12, 16), den1p.reshape(NW, P, 2),
      num1p.reshape(NW, P, 20), den2p, num2p,
      den3p.reshape(NW, P, 8), num3p.reshape(NW, P, 128),
      e1m, e3m, Wlin, bias1.reshape(1, 20), bias2.reshape(1, 1),
      blin.reshape(1, 20))

    xcat = jnp.concatenate(
        [t1[:N1].reshape(100, 17), t2[0, :N2].reshape(100, 24),
         t3[:N3].reshape(100, 17)], axis=1)
    return pl.pallas_call(
        _final_body,
        out_shape=jax.ShapeDtypeStruct((100, 7), F32),
    )(xcat, Wf, bf.reshape(1, 7))


# speedup vs baseline: 16.0725x; 1.1761x over previous
import functools

import jax
import jax.numpy as jnp
from jax import lax
from jax.experimental import pallas as pl
from jax.experimental.pallas import tpu as pltpu
from jax.experimental.pallas import tpu_sc as plsc

F32 = jnp.float32
NEG = -3e38

N1, H1, C1 = 85, 2, 10
N2 = 2400
N3, H3, C3 = 85, 8, 16
P = 104
E1 = 1536
E2 = 40960
E3 = 1536
NW = 32
E1T, E2T, E3T = E1 // NW, E2 // NW, E3 // NW
G1, G2, G3 = E1T // 16, E2T // 16, E3T // 16

SZ_XW1 = 4160
SZ_XW2 = 4832
SZ_H3 = 10880
SZ_AS = 680
SZ_AD = 832
SZ_D1 = 208
SZ_N1 = 2080
SZ_D2 = 2416
SZ_D3 = 832
SZ_N3 = 13312
SZ_MAX = 192


def _proj_body(x1, wl1, bl1, wr1, br1, x2, wl2, bl2, wr2, br2,
               x3, wp, bp, asrc, adst, att1, att2,
               xw1, xw2, h3, as_, ad_, consts):
    h = jnp.dot(x3[...], wp[...], preferred_element_type=F32) + bp[...]
    h3[...] = h
    asm = asrc[...]
    adm = adst[...]
    acols, dcols = [], []
    for hh in range(8):
        blk = h[:, hh * 16:(hh + 1) * 16]
        acols.append(jnp.sum(blk * asm[hh:hh + 1, :], axis=1, keepdims=True))
        dcols.append(jnp.sum(blk * adm[hh:hh + 1, :], axis=1, keepdims=True))
    as_[...] = jnp.concatenate(acols, axis=1)
    ad_[...] = jnp.concatenate(dcols, axis=1)
    xw1[...] = jnp.concatenate(
        [jnp.dot(x1[...], wl1[...], preferred_element_type=F32) + bl1[...],
         jnp.dot(x1[...], wr1[...], preferred_element_type=F32) + br1[...]],
        axis=1)
    xw2[...] = jnp.concatenate(
        [jnp.dot(x2[...], wl2[...], preferred_element_type=F32) + bl2[...],
         jnp.dot(x2[...], wr2[...], preferred_element_type=F32) + br2[...]],
        axis=1)
    a1 = att1[...]
    consts[...] = jnp.concatenate(
        [a1[0:1, :], a1[1:2, :], att2[...], jnp.zeros((1, 11), F32)], axis=1)


def _zero(ref, n):
    z = jnp.zeros((16,), F32)

    def body(i, c):
        base = i * 128
        for j in range(8):
            ref[pl.ds(base + j * 16, 16)] = z
        return c
    lax.fori_loop(0, n // 128, body, 0)
    for k in range(n // 128 * 128, n, 16):
        ref[pl.ds(k, 16)] = z


O_XW1 = 0
O_XW2 = 3400
O_AS = 8200
O_AD = 8880
O_H3 = 9560
O_CONST = 20440
SZ_NODES = 20472

OFF_S1 = 0
OFF_D1 = E1
OFF_S3 = 2 * E1
OFF_D3 = 3 * E1

E2R = 38400 // NW
E2L = N2 // NW


def _sc_body(nodes, edges_small, ei2,
             maxp, den1p, num1p, den2p, num2p, den3p, num3p,
             constsv, xw1v, xw2v, h3v, asv, adv,
             s1v, d1v, s2v, d2v, s3v, d3v,
             den1a, num1a, den2a, num2a, den3a, num3a,
             lbuf1, lbuf2, lbuf3, maxs, sem):
    wid = lax.axis_index("s") * 2 + lax.axis_index("c")
    z = jnp.zeros((16,), F32)

    es = edges_small
    copies = [
        pltpu.async_copy(nodes.at[pl.ds(O_CONST, 32)], constsv, sem),
        pltpu.async_copy(nodes.at[pl.ds(O_XW1, 3400)],
                         xw1v.at[pl.ds(0, 3400)], sem),
        pltpu.async_copy(nodes.at[pl.ds(O_XW2, 4800)],
                         xw2v.at[pl.ds(0, 4800)], sem),
        pltpu.async_copy(nodes.at[pl.ds(O_H3, SZ_H3)], h3v, sem),
        pltpu.async_copy(nodes.at[pl.ds(O_AS, SZ_AS)], asv, sem),
        pltpu.async_copy(nodes.at[pl.ds(O_AD, 680)],
                         adv.at[pl.ds(0, 680)], sem),
        pltpu.async_copy(es.at[pl.ds(OFF_S1 + wid * E1T, E1T)], s1v, sem),
        pltpu.async_copy(es.at[pl.ds(OFF_D1 + wid * E1T, E1T)], d1v, sem),
        pltpu.async_copy(ei2.at[0, pl.ds(wid * E2R, E2R)],
                         s2v.at[pl.ds(0, E2R)], sem),
        pltpu.async_copy(ei2.at[1, pl.ds(wid * E2R, E2R)],
                         d2v.at[pl.ds(0, E2R)], sem),
        pltpu.async_copy(es.at[pl.ds(OFF_S3 + wid * E3T, E3T)], s3v, sem),
        pltpu.async_copy(es.at[pl.ds(OFF_D3 + wid * E3T, E3T)], d3v, sem),
    ]

    iota = lax.iota(jnp.int32, 16)
    lbase = wid * E2L
    for k in range((E2T - E2R) // 16):
        idx = k * 16 + iota
        self_id = lbase + idx
        s2v[pl.ds(E2R + k * 16, 16)] = jnp.where(idx < E2L, self_id, 0)
        d2v[pl.ds(E2R + k * 16, 16)] = jnp.where(
            idx < E2L, self_id, N2 + idx - E2L)

    for k in list(range(3400, 4136, 16)) + [4144]:
        xw1v[pl.ds(k, 16)] = z
    xw2v[pl.ds(4800, 16)] = z
    xw2v[pl.ds(4816, 16)] = z
    for k in list(range(680, 808, 16)) + [816]:
        adv[pl.ds(k, 16)] = z
    _zero(den1a, SZ_D1)
    _zero(num1a, SZ_N1)
    _zero(den2a, SZ_D2)
    _zero(num2a, SZ_D2)
    _zero(den3a, SZ_D3)
    _zero(num3a, SZ_N3)

    for c in copies:
        c.wait()

    neg = jnp.full((16,), NEG, F32)
    cv0 = constsv[pl.ds(0, 16)]
    cv1 = constsv[pl.ds(16, 16)]

    def att1c(j):
        return cv0[j] if j < 16 else cv1[j - 16]

    att2 = cv1[4]

    def g2p1(i, m):
        for u in range(2):
            g = i * 2 + u
            s = s2v[pl.ds(g * 16, 16)]
            d = d2v[pl.ds(g * 16, 16)]
            xa = plsc.load_gather(xw2v, [s * 2])
            xb = plsc.load_gather(xw2v, [d * 2 + 1])
            t = xa + xb
            l = att2 * jnp.where(t > 0, t, t * 0.2)
            lbuf2[pl.ds(g * 16, 16)] = l
            m = jnp.maximum(m, l)
        return m

    m2 = lax.fori_loop(0, G2 // 2, g2p1, neg)
    maxs[pl.ds(2 * 16, 16)] = m2
    m2s = jnp.max(m2)

    def g2p2(i, c):
        for u in range(2):
            g = i * 2 + u
            l = lbuf2[pl.ds(g * 16, 16)]
            e = jnp.exp(l - m2s)
            s = s2v[pl.ds(g * 16, 16)]
            d = d2v[pl.ds(g * 16, 16)]
            xa = plsc.load_gather(xw2v, [s * 2])
            plsc.addupdate_scatter(den2a, [d], e)
            plsc.addupdate_scatter(num2a, [d], e * xa)
        return c

    lax.fori_loop(0, G2 // 2, g2p2, 0)

    m1 = [neg, neg]
    for i in range(G1):
        s = s1v[pl.ds(i * 16, 16)]
        d = d1v[pl.ds(i * 16, 16)]
        sb = s * 40
        db = d * 40
        for h in range(H1):
            acc = z
            for c in range(C1):
                xa = plsc.load_gather(xw1v, [sb + (h * 10 + c)])
                xb = plsc.load_gather(xw1v, [db + (20 + h * 10 + c)])
                t = xa + xb
                acc = acc + att1c(h * 10 + c) * jnp.where(t > 0, t, t * 0.2)
            lbuf1[pl.ds(h * E1T + i * 16, 16)] = acc
            m1[h] = jnp.maximum(m1[h], acc)
    m1s = []
    for h in range(H1):
        maxs[pl.ds(h * 16, 16)] = m1[h]
        m1s.append(jnp.max(m1[h]))
    for i in range(G1):
        s = s1v[pl.ds(i * 16, 16)]
        d = d1v[pl.ds(i * 16, 16)]
        sb = s * 40
        dn = d * 20
        dd = d * 2
        for h in range(H1):
            l = lbuf1[pl.ds(h * E1T + i * 16, 16)]
            e = jnp.exp(l - m1s[h])
            plsc.addupdate_scatter(den1a, [dd + h], e)
            for c in range(C1):
                xa = plsc.load_gather(xw1v, [sb + (h * 10 + c)])
                plsc.addupdate_scatter(num1a, [dn + (h * 10 + c)], e * xa)

    m3 = [neg] * H3
    for i in range(G3):
        s = s3v[pl.ds(i * 16, 16)]
        d = d3v[pl.ds(i * 16, 16)]
        s8 = s * 8
        d8 = d * 8
        for h in range(H3):
            xa = plsc.load_gather(asv, [s8 + h])
            xb = plsc.load_gather(adv, [d8 + h])
            t = xa + xb
            l = jnp.where(t > 0, t, t * 0.2)
            lbuf3[pl.ds(h * E3T + i * 16, 16)] = l
            m3[h] = jnp.maximum(m3[h], l)
    m3s = []
    for h in range(H3):
        maxs[pl.ds((3 + h) * 16, 16)] = m3[h]
        m3s.append(jnp.max(m3[h]))
    maxs[pl.ds(11 * 16, 16)] = z
    for i in range(G3):
        s = s3v[pl.ds(i * 16, 16)]
        d = d3v[pl.ds(i * 16, 16)]
        s128 = s * 128
        d128 = d * 128
        d8 = d * 8
        for h in range(H3):
            l = lbuf3[pl.ds(h * E3T + i * 16, 16)]
            e = jnp.exp(l - m3s[h])
            plsc.addupdate_scatter(den3a, [d8 + h], e)
            for c in range(C3):
                xa = plsc.load_gather(h3v, [s128 + (h * 16 + c)])
                plsc.addupdate_scatter(num3a, [d128 + (h * 16 + c)], e * xa)

    pltpu.sync_copy(maxs, maxp.at[wid])
    pltpu.sync_copy(den1a, den1p.at[wid])
    pltpu.sync_copy(num1a, num1p.at[wid])
    pltpu.sync_copy(den2a, den2p.at[wid])
    pltpu.sync_copy(num2a, num2p.at[wid])
    pltpu.sync_copy(den3a, den3p.at[wid])
    pltpu.sync_copy(num3a, num3p.at[wid])


_SC_OUT = [
    jax.ShapeDtypeStruct((NW, SZ_MAX), F32),
    jax.ShapeDtypeStruct((NW, SZ_D1), F32),
    jax.ShapeDtypeStruct((NW, SZ_N1), F32),
    jax.ShapeDtypeStruct((NW, SZ_D2), F32),
    jax.ShapeDtypeStruct((NW, SZ_D2), F32),
    jax.ShapeDtypeStruct((NW, SZ_D3), F32),
    jax.ShapeDtypeStruct((NW, SZ_N3), F32),
]

_SC_SCRATCH = [
    pltpu.VMEM((32,), F32),
    pltpu.VMEM((SZ_XW1,), F32),
    pltpu.VMEM((SZ_XW2,), F32),
    pltpu.VMEM((SZ_H3,), F32),
    pltpu.VMEM((SZ_AS,), F32),
    pltpu.VMEM((SZ_AD,), F32),
    pltpu.VMEM((E1T,), jnp.int32),
    pltpu.VMEM((E1T,), jnp.int32),
    pltpu.VMEM((E2T,), jnp.int32),
    pltpu.VMEM((E2T,), jnp.int32),
    pltpu.VMEM((E3T,), jnp.int32),
    pltpu.VMEM((E3T,), jnp.int32),
    pltpu.VMEM((SZ_D1,), F32),
    pltpu.VMEM((SZ_N1,), F32),
    pltpu.VMEM((SZ_D2,), F32),
    pltpu.VMEM((SZ_D2,), F32),
    pltpu.VMEM((SZ_D3,), F32),
    pltpu.VMEM((SZ_N3,), F32),
    pltpu.VMEM((H1 * E1T,), F32),
    pltpu.VMEM((E2T,), F32),
    pltpu.VMEM((H3 * E3T,), F32),
    pltpu.VMEM((SZ_MAX,), F32),
    pltpu.SemaphoreType.DMA,
]


@functools.cache
def _sc_kernel():
    return functools.partial(
        pl.kernel,
        out_type=_SC_OUT,
        mesh=plsc.VectorSubcoreMesh(
            core_axis_name="c", subcore_axis_name="s",
            num_cores=2, num_subcores=16),
        scratch_types=_SC_SCRATCH,
        compiler_params=pltpu.CompilerParams(
            use_tc_tiling_on_sc=False, needs_layout_passes=False),
    )(_sc_body)


def _sc_call(*args):
    return _sc_kernel()(*args)


def _combine_body(maxp3, d1p, n1p, d2p, n2p, d3p, n3p,
                  e1, e3, wlin, b1, b2, blin, t1o, t2o, t3o):
    mt = jnp.max(maxp3[...], axis=2)
    gm = jnp.max(mt, axis=0)
    sc = jnp.exp(mt - gm[None, :])
    e1m = e1[...]
    e3m = e3[...]

    sc1 = sc[:, 0:2]
    den1 = jnp.sum(d1p[...] * sc1[:, None, :], axis=0)
    s1pat = jnp.dot(sc1, e1m, preferred_element_type=F32)
    num1 = jnp.sum(n1p[...] * s1pat[:, None, :], axis=0)
    dex1 = jnp.dot(den1, e1m, preferred_element_type=F32)
    t1o[...] = jnp.maximum(num1 / (dex1 + 1e-16) + b1[...], 0.0)

    sc2 = sc[:, 2:3]
    den2 = jnp.sum(d2p[...] * sc2, axis=0, keepdims=True)
    num2 = jnp.sum(n2p[...] * sc2, axis=0, keepdims=True)
    t2o[...] = jnp.maximum(num2 / (den2 + 1e-16) + b2[...], 0.0)

    sc3 = sc[:, 3:11]
    den3 = jnp.sum(d3p[...] * sc3[:, None, :], axis=0)
    s3pat = jnp.dot(sc3, e3m, preferred_element_type=F32)
    num3 = jnp.sum(n3p[...] * s3pat[:, None, :], axis=0)
    dex3 = jnp.dot(den3, e3m, preferred_element_type=F32)
    pre = jnp.maximum(num3 / (dex3 + 1e-16), 0.0)
    o3 = jnp.dot(pre, wlin[...], preferred_element_type=F32) + blin[...]
    t3o[...] = jnp.maximum(o3, 0.0)


def _final_body(x_ref, w_ref, bf_ref, o_ref):
    o_ref[...] = jnp.dot(x_ref[...], w_ref[...],
                         preferred_element_type=F32) + bf_ref[...]


def kernel(x1, edge_index1, x2, edge_index2, x3, edge_index3, Wl1, bl1, Wr1, br1, att1, bias1, Wl2, bl2, Wr2, br2, att2, bias2, Wp, bp, asrc, adst, Wk, bk, q, Wlin, blin, Wf, bf):
    i32 = edge_index1.dtype

    xw1, xw2, h3, as_, ad_, consts = pl.pallas_call(
        _proj_body,
        out_shape=[
            jax.ShapeDtypeStruct((N1, 40), F32),
            jax.ShapeDtypeStruct((N2, 2), F32),
            jax.ShapeDtypeStruct((N3, 128), F32),
            jax.ShapeDtypeStruct((N3, 8), F32),
            jax.ShapeDtypeStruct((N3, 8), F32),
            jax.ShapeDtypeStruct((1, 32), F32),
        ],
    )(x1, Wl1, bl1.reshape(1, 20), Wr1, br1.reshape(1, 20),
      x2, Wl2, bl2.reshape(1, 1), Wr2, br2.reshape(1, 1),
      x3, Wp, bp.reshape(1, 128), asrc, adst, att1, att2)

    lp1 = jnp.arange(N1, dtype=i32)
    zk = lambda k: jnp.zeros((k,), i32)
    dump = lambda n, k: (n + jnp.arange(k, dtype=i32) % 16).astype(i32)
    p1 = E1 - 1360 - N1
    p3 = E3 - 1360
    edges_small = jnp.concatenate([
        edge_index1[0], lp1, zk(p1),
        edge_index1[1], lp1, dump(N1, p1),
        edge_index3[0], zk(p3),
        edge_index3[1], dump(N3, p3),
    ])
    nodes = jnp.concatenate([
        xw1.reshape(N1 * 40), xw2.reshape(N2 * 2),
        as_.reshape(N3 * 8), ad_.reshape(N3 * 8),
        h3.reshape(SZ_H3), consts.reshape(32)])

    maxp, den1p, num1p, den2p, num2p, den3p, num3p = _sc_call(
        nodes, edges_small, edge_index2)

    e1m = jnp.repeat(jnp.eye(2, dtype=F32), 10, axis=1)
    e3m = jnp.repeat(jnp.eye(8, dtype=F32), 16, axis=1)
    t1, t2, t3 = pl.pallas_call(
        _combine_body,
        out_shape=[
            jax.ShapeDtypeStruct((P, 20), F32),
            jax.ShapeDtypeStruct((1, SZ_D2), F32),
            jax.ShapeDtypeStruct((P, 20), F32),
        ],
    )(maxp.reshape(NW, 12, 16), den1p.reshape(NW, P, 2),
      num1p.reshape(NW, P, 20), den2p, num2p,
      den3p.reshape(NW, P, 8), num3p.reshape(NW, P, 128),
      e1m, e3m, Wlin, bias1.reshape(1, 20), bias2.reshape(1, 1),
      blin.reshape(1, 20))

    xcat = jnp.concatenate(
        [t1[:N1].reshape(100, 17), t2[0, :N2].reshape(100, 24),
         t3[:N3].reshape(100, 17)], axis=1)
    return pl.pallas_call(
        _final_body,
        out_shape=jax.ShapeDtypeStruct((100, 7), F32),
    )(xcat, Wf, bf.reshape(1, 7))


# speedup vs baseline: 16.1777x; 1.0065x over previous
import functools

import jax
import jax.numpy as jnp
from jax import lax
from jax.experimental import pallas as pl
from jax.experimental.pallas import tpu as pltpu
from jax.experimental.pallas import tpu_sc as plsc

F32 = jnp.float32
NEG = -3e38

N1, H1, C1 = 85, 2, 10
N2 = 2400
N3, H3, C3 = 85, 8, 16
P = 104
E1 = 1536
E2 = 40960
E3 = 1536
NW = 32
E1T, E2T, E3T = E1 // NW, E2 // NW, E3 // NW
G1, G2, G3 = E1T // 16, E2T // 16, E3T // 16

SZ_XW1 = 4160
SZ_XW2 = 4832
SZ_H3 = 10880
SZ_AS = 680
SZ_AD = 832
SZ_D1 = 208
SZ_N1 = 2080
SZ_D2 = 2416
SZ_D3 = 832
SZ_N3 = 13312
SZ_MAX = 192


def _proj_body(x1, wl1, bl1, wr1, br1, x2, wl2, bl2, wr2, br2,
               x3, wp, bp, asrc, adst, att1, att2,
               xw1, xw2, h3, as_, ad_, consts):
    h = jnp.dot(x3[...], wp[...], preferred_element_type=F32) + bp[...]
    h3[...] = h
    asm = asrc[...]
    adm = adst[...]
    acols, dcols = [], []
    for hh in range(8):
        blk = h[:, hh * 16:(hh + 1) * 16]
        acols.append(jnp.sum(blk * asm[hh:hh + 1, :], axis=1, keepdims=True))
        dcols.append(jnp.sum(blk * adm[hh:hh + 1, :], axis=1, keepdims=True))
    as_[...] = jnp.concatenate(acols, axis=1)
    ad_[...] = jnp.concatenate(dcols, axis=1)
    xw1[...] = jnp.concatenate(
        [jnp.dot(x1[...], wl1[...], preferred_element_type=F32) + bl1[...],
         jnp.dot(x1[...], wr1[...], preferred_element_type=F32) + br1[...]],
        axis=1)
    xw2[...] = jnp.concatenate(
        [jnp.dot(x2[...], wl2[...], preferred_element_type=F32) + bl2[...],
         jnp.dot(x2[...], wr2[...], preferred_element_type=F32) + br2[...]],
        axis=1)
    a1 = att1[...]
    consts[...] = jnp.concatenate(
        [a1[0:1, :], a1[1:2, :], att2[...], jnp.zeros((1, 11), F32)], axis=1)


def _zero(ref, n):
    z = jnp.zeros((16,), F32)

    def body(i, c):
        base = i * 128
        for j in range(8):
            ref[pl.ds(base + j * 16, 16)] = z
        return c
    lax.fori_loop(0, n // 128, body, 0)
    for k in range(n // 128 * 128, n, 16):
        ref[pl.ds(k, 16)] = z


O_XW1 = 0
O_XW2 = 3400
O_AS = 8200
O_AD = 8880
O_H3 = 9560
O_CONST = 20440
SZ_NODES = 20472

OFF_S1 = 0
OFF_D1 = E1
OFF_S3 = 2 * E1
OFF_D3 = 3 * E1

E2R = 38400 // NW
E2L = N2 // NW


_RED = [
    (0, 0, 208, 0, 0),
    (1, 0, 1040, 1, 0),
    (1, 1040, 1040, 1, 1040),
    (2, 0, 1216, 2, 0),
    (2, 1216, 1200, 2, 1216),
    (3, 0, 1216, 3, 0),
    (3, 1216, 1200, 3, 1216),
    (4, 0, 832, 4, 0),
] + [(5, k * 1664, 1664, 5, k * 1664) for k in range(8)]


def _sc_body(nodes, edges_small, ei2,
             maxp, den1p, num1p, den2p, num2p, den3p, num3p,
             constsv, xw1v, xw2v, h3v, asv, adv,
             s1v, d1v, s2v, d2v, s3v, d3v,
             den1a, num1a, den2a, num2a, den3a, num3a,
             lbuf1, lbuf2, lbuf3, maxs, maxall, redv, resv,
             maxslab, slab1, slab2, slab3, slab4, slab5, slab6, sem):
    cid = lax.axis_index("c")
    sid = lax.axis_index("s")
    wid = sid * 2 + cid
    z = jnp.zeros((16,), F32)

    es = edges_small
    copies = [
        pltpu.async_copy(nodes.at[pl.ds(O_CONST, 32)], constsv, sem),
        pltpu.async_copy(nodes.at[pl.ds(O_XW1, 3400)],
                         xw1v.at[pl.ds(0, 3400)], sem),
        pltpu.async_copy(nodes.at[pl.ds(O_XW2, 4800)],
                         xw2v.at[pl.ds(0, 4800)], sem),
        pltpu.async_copy(nodes.at[pl.ds(O_H3, SZ_H3)], h3v, sem),
        pltpu.async_copy(nodes.at[pl.ds(O_AS, SZ_AS)], asv, sem),
        pltpu.async_copy(nodes.at[pl.ds(O_AD, 680)],
                         adv.at[pl.ds(0, 680)], sem),
        pltpu.async_copy(es.at[pl.ds(OFF_S1 + wid * E1T, E1T)], s1v, sem),
        pltpu.async_copy(es.at[pl.ds(OFF_D1 + wid * E1T, E1T)], d1v, sem),
        pltpu.async_copy(ei2.at[0, pl.ds(wid * E2R, E2R)],
                         s2v.at[pl.ds(0, E2R)], sem),
        pltpu.async_copy(ei2.at[1, pl.ds(wid * E2R, E2R)],
                         d2v.at[pl.ds(0, E2R)], sem),
        pltpu.async_copy(es.at[pl.ds(OFF_S3 + wid * E3T, E3T)], s3v, sem),
        pltpu.async_copy(es.at[pl.ds(OFF_D3 + wid * E3T, E3T)], d3v, sem),
    ]

    iota = lax.iota(jnp.int32, 16)
    lbase = wid * E2L
    for k in range((E2T - E2R) // 16):
        idx = k * 16 + iota
        self_id = lbase + idx
        s2v[pl.ds(E2R + k * 16, 16)] = jnp.where(idx < E2L, self_id, 0)
        d2v[pl.ds(E2R + k * 16, 16)] = jnp.where(
            idx < E2L, self_id, N2 + idx - E2L)

    for k in list(range(3400, 4136, 16)) + [4144]:
        xw1v[pl.ds(k, 16)] = z
    xw2v[pl.ds(4800, 16)] = z
    xw2v[pl.ds(4816, 16)] = z
    for k in list(range(680, 808, 16)) + [816]:
        adv[pl.ds(k, 16)] = z
    _zero(den1a, SZ_D1)
    _zero(num1a, SZ_N1)
    _zero(den2a, SZ_D2)
    _zero(num2a, SZ_D2)
    _zero(den3a, SZ_D3)
    _zero(num3a, SZ_N3)

    for c in copies:
        c.wait()

    neg = jnp.full((16,), NEG, F32)
    cv0 = constsv[pl.ds(0, 16)]
    cv1 = constsv[pl.ds(16, 16)]

    def att1c(j):
        return cv0[j] if j < 16 else cv1[j - 16]

    att2 = cv1[4]

    def g2p1(i, m):
        for u in range(2):
            g = i * 2 + u
            s = s2v[pl.ds(g * 16, 16)]
            d = d2v[pl.ds(g * 16, 16)]
            xa = plsc.load_gather(xw2v, [s * 2])
            xb = plsc.load_gather(xw2v, [d * 2 + 1])
            t = xa + xb
            l = att2 * jnp.where(t > 0, t, t * 0.2)
            lbuf2[pl.ds(g * 16, 16)] = l
            m = jnp.maximum(m, l)
        return m

    m2 = lax.fori_loop(0, G2 // 2, g2p1, neg)
    maxs[pl.ds(2 * 16, 16)] = m2

    m1 = [neg, neg]
    for i in range(G1):
        s = s1v[pl.ds(i * 16, 16)]
        d = d1v[pl.ds(i * 16, 16)]
        sb = s * 40
        db = d * 40
        for h in range(H1):
            acc = z
            for c in range(C1):
                xa = plsc.load_gather(xw1v, [sb + (h * 10 + c)])
                xb = plsc.load_gather(xw1v, [db + (20 + h * 10 + c)])
                t = xa + xb
                acc = acc + att1c(h * 10 + c) * jnp.where(t > 0, t, t * 0.2)
            lbuf1[pl.ds(h * E1T + i * 16, 16)] = acc
            m1[h] = jnp.maximum(m1[h], acc)
    for h in range(H1):
        maxs[pl.ds(h * 16, 16)] = m1[h]

    m3 = [neg] * H3
    for i in range(G3):
        s = s3v[pl.ds(i * 16, 16)]
        d = d3v[pl.ds(i * 16, 16)]
        s8 = s * 8
        d8 = d * 8
        for h in range(H3):
            xa = plsc.load_gather(asv, [s8 + h])
            xb = plsc.load_gather(adv, [d8 + h])
            t = xa + xb
            l = jnp.where(t > 0, t, t * 0.2)
            lbuf3[pl.ds(h * E3T + i * 16, 16)] = l
            m3[h] = jnp.maximum(m3[h], l)
    for h in range(H3):
        maxs[pl.ds((3 + h) * 16, 16)] = m3[h]
    maxs[pl.ds(11 * 16, 16)] = z

    pltpu.sync_copy(maxs, maxslab.at[sid])
    plsc.subcore_barrier()
    pltpu.sync_copy(maxslab, maxall)
    shift = []
    for slot in range(11):
        v = maxall[0, pl.ds(slot * 16, 16)]
        for r in range(1, 16):
            v = jnp.maximum(v, maxall[r, pl.ds(slot * 16, 16)])
        maxs[pl.ds(slot * 16, 16)] = v
        shift.append(jnp.max(v))
    m1s = shift[0:2]
    m2s = shift[2]
    m3s = shift[3:11]

    def g2p2(i, c):
        for u in range(2):
            g = i * 2 + u
            l = lbuf2[pl.ds(g * 16, 16)]
            e = jnp.exp(l - m2s)
            s = s2v[pl.ds(g * 16, 16)]
            d = d2v[pl.ds(g * 16, 16)]
            xa = plsc.load_gather(xw2v, [s * 2])
            plsc.addupdate_scatter(den2a, [d], e)
            plsc.addupdate_scatter(num2a, [d], e * xa)
        return c

    lax.fori_loop(0, G2 // 2, g2p2, 0)

    for i in range(G1):
        s = s1v[pl.ds(i * 16, 16)]
        d = d1v[pl.ds(i * 16, 16)]
        sb = s * 40
        dn = d * 20
        dd = d * 2
        for h in range(H1):
            l = lbuf1[pl.ds(h * E1T + i * 16, 16)]
            e = jnp.exp(l - m1s[h])
            plsc.addupdate_scatter(den1a, [dd + h], e)
            for c in range(C1):
                xa = plsc.load_gather(xw1v, [sb + (h * 10 + c)])
                plsc.addupdate_scatter(num1a, [dn + (h * 10 + c)], e * xa)

    for i in range(G3):
        s = s3v[pl.ds(i * 16, 16)]
        d = d3v[pl.ds(i * 16, 16)]
        s128 = s * 128
        d128 = d * 128
        d8 = d * 8
        for h in range(H3):
            l = lbuf3[pl.ds(h * E3T + i * 16, 16)]
            e = jnp.exp(l - m3s[h])
            plsc.addupdate_scatter(den3a, [d8 + h], e)
            for c in range(C3):
                xa = plsc.load_gather(h3v, [s128 + (h * 16 + c)])
                plsc.addupdate_scatter(num3a, [d128 + (h * 16 + c)], e * xa)

    stage = [
        pltpu.async_copy(den1a, slab1.at[sid], sem),
        pltpu.async_copy(num1a, slab2.at[sid], sem),
        pltpu.async_copy(den2a, slab3.at[sid], sem),
        pltpu.async_copy(num2a, slab4.at[sid], sem),
        pltpu.async_copy(den3a, slab5.at[sid], sem),
        pltpu.async_copy(num3a, slab6.at[sid], sem),
    ]
    for c in stage:
        c.wait()
    plsc.subcore_barrier()

    @pl.when(sid == 0)
    def _():
        pltpu.sync_copy(maxs, maxp.at[cid])

    slabs = (slab1, slab2, slab3, slab4, slab5, slab6)
    outs = (den1p, num1p, den2p, num2p, den3p, num3p)
    for owner, (si, off, ln, oi, ooff) in enumerate(_RED):
        @pl.when(sid == owner)
        def _(si=si, off=off, ln=ln, oi=oi, ooff=ooff):
            slab = slabs[si]
            pulls = [pltpu.async_copy(slab.at[r, pl.ds(off, ln)],
                                      redv.at[r, pl.ds(0, ln)], sem)
                     for r in range(16)]
            for c in pulls:
                c.wait()

            def chunk(j, c):
                v = redv[0, pl.ds(j * 16, 16)]
                for r in range(1, 16):
                    v = v + redv[r, pl.ds(j * 16, 16)]
                resv[pl.ds(j * 16, 16)] = v
                return c

            lax.fori_loop(0, ln // 16, chunk, 0)
            pltpu.sync_copy(resv.at[pl.ds(0, ln)],
                            outs[oi].at[cid, pl.ds(ooff, ln)])


_SC_OUT = [
    jax.ShapeDtypeStruct((2, SZ_MAX), F32),
    jax.ShapeDtypeStruct((2, SZ_D1), F32),
    jax.ShapeDtypeStruct((2, SZ_N1), F32),
    jax.ShapeDtypeStruct((2, SZ_D2), F32),
    jax.ShapeDtypeStruct((2, SZ_D2), F32),
    jax.ShapeDtypeStruct((2, SZ_D3), F32),
    jax.ShapeDtypeStruct((2, SZ_N3), F32),
]

_SC_SCRATCH = [
    pltpu.VMEM((32,), F32),
    pltpu.VMEM((SZ_XW1,), F32),
    pltpu.VMEM((SZ_XW2,), F32),
    pltpu.VMEM((SZ_H3,), F32),
    pltpu.VMEM((SZ_AS,), F32),
    pltpu.VMEM((SZ_AD,), F32),
    pltpu.VMEM((E1T,), jnp.int32),
    pltpu.VMEM((E1T,), jnp.int32),
    pltpu.VMEM((E2T,), jnp.int32),
    pltpu.VMEM((E2T,), jnp.int32),
    pltpu.VMEM((E3T,), jnp.int32),
    pltpu.VMEM((E3T,), jnp.int32),
    pltpu.VMEM((SZ_D1,), F32),
    pltpu.VMEM((SZ_N1,), F32),
    pltpu.VMEM((SZ_D2,), F32),
    pltpu.VMEM((SZ_D2,), F32),
    pltpu.VMEM((SZ_D3,), F32),
    pltpu.VMEM((SZ_N3,), F32),
    pltpu.VMEM((H1 * E1T,), F32),
    pltpu.VMEM((E2T,), F32),
    pltpu.VMEM((H3 * E3T,), F32),
    pltpu.VMEM((SZ_MAX,), F32),
    pltpu.VMEM((16, SZ_MAX), F32),
    pltpu.VMEM((16, 1664), F32),
    pltpu.VMEM((1664,), F32),
    pltpu.VMEM_SHARED((16, SZ_MAX), F32),
    pltpu.VMEM_SHARED((16, SZ_D1), F32),
    pltpu.VMEM_SHARED((16, SZ_N1), F32),
    pltpu.VMEM_SHARED((16, SZ_D2), F32),
    pltpu.VMEM_SHARED((16, SZ_D2), F32),
    pltpu.VMEM_SHARED((16, SZ_D3), F32),
    pltpu.VMEM_SHARED((16, SZ_N3), F32),
    pltpu.SemaphoreType.DMA,
]


@functools.cache
def _sc_kernel():
    return functools.partial(
        pl.kernel,
        out_type=_SC_OUT,
        mesh=plsc.VectorSubcoreMesh(
            core_axis_name="c", subcore_axis_name="s",
            num_cores=2, num_subcores=16),
        scratch_types=_SC_SCRATCH,
        compiler_params=pltpu.CompilerParams(
            use_tc_tiling_on_sc=False, needs_layout_passes=False),
    )(_sc_body)


def _sc_call(*args):
    return _sc_kernel()(*args)


def _combine_body(maxp3, d1p, n1p, d2p, n2p, d3p, n3p,
                  e1, e3, wlin, b1, b2, blin, t1o, t2o, t3o):
    mt = jnp.max(maxp3[...], axis=2)
    gm = jnp.max(mt, axis=0)
    sc = jnp.exp(mt - gm[None, :])
    e1m = e1[...]
    e3m = e3[...]

    sc1 = sc[:, 0:2]
    den1 = jnp.sum(d1p[...] * sc1[:, None, :], axis=0)
    s1pat = jnp.dot(sc1, e1m, preferred_element_type=F32)
    num1 = jnp.sum(n1p[...] * s1pat[:, None, :], axis=0)
    dex1 = jnp.dot(den1, e1m, preferred_element_type=F32)
    t1o[...] = jnp.maximum(num1 / (dex1 + 1e-16) + b1[...], 0.0)

    sc2 = sc[:, 2:3]
    den2 = jnp.sum(d2p[...] * sc2, axis=0, keepdims=True)
    num2 = jnp.sum(n2p[...] * sc2, axis=0, keepdims=True)
    t2o[...] = jnp.maximum(num2 / (den2 + 1e-16) + b2[...], 0.0)

    sc3 = sc[:, 3:11]
    den3 = jnp.sum(d3p[...] * sc3[:, None, :], axis=0)
    s3pat = jnp.dot(sc3, e3m, preferred_element_type=F32)
    num3 = jnp.sum(n3p[...] * s3pat[:, None, :], axis=0)
    dex3 = jnp.dot(den3, e3m, preferred_element_type=F32)
    pre = jnp.maximum(num3 / (dex3 + 1e-16), 0.0)
    o3 = jnp.dot(pre, wlin[...], preferred_element_type=F32) + blin[...]
    t3o[...] = jnp.maximum(o3, 0.0)


def _final_body(x_ref, w_ref, bf_ref, o_ref):
    o_ref[...] = jnp.dot(x_ref[...], w_ref[...],
                         preferred_element_type=F32) + bf_ref[...]


def kernel(x1, edge_index1, x2, edge_index2, x3, edge_index3, Wl1, bl1, Wr1, br1, att1, bias1, Wl2, bl2, Wr2, br2, att2, bias2, Wp, bp, asrc, adst, Wk, bk, q, Wlin, blin, Wf, bf):
    i32 = edge_index1.dtype

    xw1, xw2, h3, as_, ad_, consts = pl.pallas_call(
        _proj_body,
        out_shape=[
            jax.ShapeDtypeStruct((N1, 40), F32),
            jax.ShapeDtypeStruct((N2, 2), F32),
            jax.ShapeDtypeStruct((N3, 128), F32),
            jax.ShapeDtypeStruct((N3, 8), F32),
            jax.ShapeDtypeStruct((N3, 8), F32),
            jax.ShapeDtypeStruct((1, 32), F32),
        ],
    )(x1, Wl1, bl1.reshape(1, 20), Wr1, br1.reshape(1, 20),
      x2, Wl2, bl2.reshape(1, 1), Wr2, br2.reshape(1, 1),
      x3, Wp, bp.reshape(1, 128), asrc, adst, att1, att2)

    lp1 = jnp.arange(N1, dtype=i32)
    zk = lambda k: jnp.zeros((k,), i32)
    dump = lambda n, k: (n + jnp.arange(k, dtype=i32) % 16).astype(i32)
    p1 = E1 - 1360 - N1
    p3 = E3 - 1360
    edges_small = jnp.concatenate([
        edge_index1[0], lp1, zk(p1),
        edge_index1[1], lp1, dump(N1, p1),
        edge_index3[0], zk(p3),
        edge_index3[1], dump(N3, p3),
    ])
    nodes = jnp.concatenate([
        xw1.reshape(N1 * 40), xw2.reshape(N2 * 2),
        as_.reshape(N3 * 8), ad_.reshape(N3 * 8),
        h3.reshape(SZ_H3), consts.reshape(32)])

    maxp, den1p, num1p, den2p, num2p, den3p, num3p = _sc_call(
        nodes, edges_small, edge_index2)

    e1m = jnp.repeat(jnp.eye(2, dtype=F32), 10, axis=1)
    e3m = jnp.repeat(jnp.eye(8, dtype=F32), 16, axis=1)
    t1, t2, t3 = pl.pallas_call(
        _combine_body,
        out_shape=[
            jax.ShapeDtypeStruct((P, 20), F32),
            jax.ShapeDtypeStruct((1, SZ_D2), F32),
            jax.ShapeDtypeStruct((P, 20), F32),
        ],
    )(maxp.reshape(2, 12, 16), den1p.reshape(2, P, 2),
      num1p.reshape(2, P, 20), den2p, num2p,
      den3p.reshape(2, P, 8), num3p.reshape(2, P, 128),
      e1m, e3m, Wlin, bias1.reshape(1, 20), bias2.reshape(1, 1),
      blin.reshape(1, 20))

    xcat = jnp.concatenate(
        [t1[:N1].reshape(100, 17), t2[0, :N2].reshape(100, 24),
         t3[:N3].reshape(100, 17)], axis=1)
    return pl.pallas_call(
        _final_body,
        out_shape=jax.ShapeDtypeStruct((100, 7), F32),
    )(xcat, Wf, bf.reshape(1, 7))


# speedup vs baseline: 18.4118x; 1.1381x over previous
import functools

import jax
import jax.numpy as jnp
from jax import lax
from jax.experimental import pallas as pl
from jax.experimental.pallas import tpu as pltpu
from jax.experimental.pallas import tpu_sc as plsc

F32 = jnp.float32
NEG = -3e38

N1, H1, C1 = 85, 2, 10
N2 = 2400
N3, H3, C3 = 85, 8, 16
P = 104
E1 = 1536
E2 = 40960
E3 = 1536
NW = 32
E1T, E2T, E3T = E1 // NW, E2 // NW, E3 // NW
G1, G2, G3 = E1T // 16, E2T // 16, E3T // 16

SZ_XW1 = 4160
SZ_XW2 = 4832
SZ_H3 = 10880
SZ_AS = 680
SZ_AD = 832
SZ_D1 = 208
SZ_N1 = 2080
SZ_D2 = 2416
SZ_D3 = 832
SZ_N3 = 13312
SZ_MAX = 192


def _proj_body(x1, wl1, bl1, wr1, br1, x2, wl2, bl2, wr2, br2,
               x3, wp, bp, asrc, adst, att1, att2,
               xw1, xw2, h3, as_, ad_, consts):
    h = jnp.dot(x3[...], wp[...], preferred_element_type=F32) + bp[...]
    h3[...] = h
    asm = asrc[...]
    adm = adst[...]
    acols, dcols = [], []
    for hh in range(8):
        blk = h[:, hh * 16:(hh + 1) * 16]
        acols.append(jnp.sum(blk * asm[hh:hh + 1, :], axis=1, keepdims=True))
        dcols.append(jnp.sum(blk * adm[hh:hh + 1, :], axis=1, keepdims=True))
    as_[...] = jnp.concatenate(acols, axis=1)
    ad_[...] = jnp.concatenate(dcols, axis=1)
    xw1[...] = jnp.concatenate(
        [jnp.dot(x1[...], wl1[...], preferred_element_type=F32) + bl1[...],
         jnp.dot(x1[...], wr1[...], preferred_element_type=F32) + br1[...]],
        axis=1)
    xw2[...] = jnp.concatenate(
        [jnp.dot(x2[...], wl2[...], preferred_element_type=F32) + bl2[...],
         jnp.dot(x2[...], wr2[...], preferred_element_type=F32) + br2[...]],
        axis=1)
    a1 = att1[...]
    consts[...] = jnp.concatenate(
        [a1[0:1, :], a1[1:2, :], att2[...], jnp.zeros((1, 11), F32)], axis=1)


def _zero(ref, n):
    z = jnp.zeros((16,), F32)

    def body(i, c):
        base = i * 128
        for j in range(8):
            ref[pl.ds(base + j * 16, 16)] = z
        return c
    lax.fori_loop(0, n // 128, body, 0)
    for k in range(n // 128 * 128, n, 16):
        ref[pl.ds(k, 16)] = z


O_XW1 = 0
O_XW2 = 3400
O_AS = 8200
O_AD = 8880
O_H3 = 9560
O_CONST = 20440
SZ_NODES = 20472

OFF_S1 = 0
OFF_D1 = E1
OFF_S3 = 2 * E1
OFF_D3 = 3 * E1

E2R = 38400 // NW
E2L = N2 // NW


S_MAXR = 0
S_PK = 1536
S_N3 = 14848
S_D2 = 28160
S_N2 = 30592
SZ_ACC = 33024
SZ_PK = 13312
SZ_D2P = 2432

B_PK = 0
B_N3 = 13312
B_D2 = 26624
B_N2 = 29056
SZ_A = 31488
CHUNK = SZ_A // 16


def _sc_body(nodes, edges_small, ei2, accp,
             constsv, xw1v, xw2v, h3v, asv, adv,
             s1v, d1v, s2v, d2v, s3v, d3v, acca,
             lbuf1, lbuf2, lbuf3, maxs, maxall, redv, resv, maxrowv,
             maxslab, slab, sem):
    cid = lax.axis_index("c")
    sid = lax.axis_index("s")
    wid = sid * 2 + cid
    z = jnp.zeros((16,), F32)

    es = edges_small
    copies = [
        pltpu.async_copy(nodes.at[pl.ds(O_CONST, 32)], constsv, sem),
        pltpu.async_copy(nodes.at[pl.ds(O_XW1, 3400)],
                         xw1v.at[pl.ds(0, 3400)], sem),
        pltpu.async_copy(nodes.at[pl.ds(O_XW2, 4800)],
                         xw2v.at[pl.ds(0, 4800)], sem),
        pltpu.async_copy(nodes.at[pl.ds(O_H3, SZ_H3)], h3v, sem),
        pltpu.async_copy(nodes.at[pl.ds(O_AS, SZ_AS)], asv, sem),
        pltpu.async_copy(nodes.at[pl.ds(O_AD, 680)],
                         adv.at[pl.ds(0, 680)], sem),
        pltpu.async_copy(es.at[pl.ds(OFF_S1 + wid * E1T, E1T)], s1v, sem),
        pltpu.async_copy(es.at[pl.ds(OFF_D1 + wid * E1T, E1T)], d1v, sem),
        pltpu.async_copy(ei2.at[0, pl.ds(wid * E2R, E2R)],
                         s2v.at[pl.ds(0, E2R)], sem),
        pltpu.async_copy(ei2.at[1, pl.ds(wid * E2R, E2R)],
                         d2v.at[pl.ds(0, E2R)], sem),
        pltpu.async_copy(es.at[pl.ds(OFF_S3 + wid * E3T, E3T)], s3v, sem),
        pltpu.async_copy(es.at[pl.ds(OFF_D3 + wid * E3T, E3T)], d3v, sem),
    ]

    iota = lax.iota(jnp.int32, 16)
    lbase = wid * E2L
    for k in range((E2T - E2R) // 16):
        idx = k * 16 + iota
        self_id = lbase + idx
        s2v[pl.ds(E2R + k * 16, 16)] = jnp.where(idx < E2L, self_id, 0)
        d2v[pl.ds(E2R + k * 16, 16)] = jnp.where(
            idx < E2L, self_id, N2 + idx - E2L)

    for k in list(range(3400, 4136, 16)) + [4144]:
        xw1v[pl.ds(k, 16)] = z
    xw2v[pl.ds(4800, 16)] = z
    xw2v[pl.ds(4816, 16)] = z
    for k in list(range(680, 808, 16)) + [816]:
        adv[pl.ds(k, 16)] = z
    _zero(acca, SZ_A)

    for c in copies:
        c.wait()

    neg = jnp.full((16,), NEG, F32)
    cv0 = constsv[pl.ds(0, 16)]
    cv1 = constsv[pl.ds(16, 16)]

    def att1c(j):
        return cv0[j] if j < 16 else cv1[j - 16]

    att2 = cv1[4]

    def g2p1(i, m):
        for u in range(2):
            g = i * 2 + u
            s = s2v[pl.ds(g * 16, 16)]
            d = d2v[pl.ds(g * 16, 16)]
            xa = plsc.load_gather(xw2v, [s * 2])
            xb = plsc.load_gather(xw2v, [d * 2 + 1])
            t = xa + xb
            l = att2 * jnp.where(t > 0, t, t * 0.2)
            lbuf2[pl.ds(g * 16, 16)] = l
            m = jnp.maximum(m, l)
        return m

    m2 = lax.fori_loop(0, G2 // 2, g2p1, neg)
    maxs[pl.ds(2 * 16, 16)] = m2

    m1 = [neg, neg]
    for i in range(G1):
        s = s1v[pl.ds(i * 16, 16)]
        d = d1v[pl.ds(i * 16, 16)]
        sb = s * 40
        db = d * 40
        for h in range(H1):
            acc = z
            for c in range(C1):
                xa = plsc.load_gather(xw1v, [sb + (h * 10 + c)])
                xb = plsc.load_gather(xw1v, [db + (20 + h * 10 + c)])
                t = xa + xb
                acc = acc + att1c(h * 10 + c) * jnp.where(t > 0, t, t * 0.2)
            lbuf1[pl.ds(h * E1T + i * 16, 16)] = acc
            m1[h] = jnp.maximum(m1[h], acc)
    for h in range(H1):
        maxs[pl.ds(h * 16, 16)] = m1[h]

    m3 = [neg] * H3
    for i in range(G3):
        s = s3v[pl.ds(i * 16, 16)]
        d = d3v[pl.ds(i * 16, 16)]
        s8 = s * 8
        d8 = d * 8
        for h in range(H3):
            xa = plsc.load_gather(asv, [s8 + h])
            xb = plsc.load_gather(adv, [d8 + h])
            t = xa + xb
            l = jnp.where(t > 0, t, t * 0.2)
            lbuf3[pl.ds(h * E3T + i * 16, 16)] = l
            m3[h] = jnp.maximum(m3[h], l)
    for h in range(H3):
        maxs[pl.ds((3 + h) * 16, 16)] = m3[h]
    maxs[pl.ds(11 * 16, 16)] = z

    pltpu.sync_copy(maxs, maxslab.at[sid])
    plsc.subcore_barrier()
    pltpu.sync_copy(maxslab, maxall)
    shift = []
    for slot in range(11):
        v = maxall[0, pl.ds(slot * 16, 16)]
        for r in range(1, 16):
            v = jnp.maximum(v, maxall[r, pl.ds(slot * 16, 16)])
        maxs[pl.ds(slot * 16, 16)] = v
        shift.append(jnp.max(v))
    m1s = shift[0:2]
    m2s = shift[2]
    m3s = shift[3:11]

    def g2p2(i, c):
        for u in range(2):
            g = i * 2 + u
            l = lbuf2[pl.ds(g * 16, 16)]
            e = jnp.exp(l - m2s)
            s = s2v[pl.ds(g * 16, 16)]
            d = d2v[pl.ds(g * 16, 16)]
            xa = plsc.load_gather(xw2v, [s * 2])
            plsc.addupdate_scatter(acca, [d + B_D2], e)
            plsc.addupdate_scatter(acca, [d + B_N2], e * xa)
        return c

    lax.fori_loop(0, G2 // 2, g2p2, 0)

    for i in range(G1):
        s = s1v[pl.ds(i * 16, 16)]
        d = d1v[pl.ds(i * 16, 16)]
        sb = s * 40
        dn = d * 128
        for h in range(H1):
            l = lbuf1[pl.ds(h * E1T + i * 16, 16)]
            e = jnp.exp(l - m1s[h])
            plsc.addupdate_scatter(acca, [dn + (20 + h)], e)
            for c in range(C1):
                xa = plsc.load_gather(xw1v, [sb + (h * 10 + c)])
                plsc.addupdate_scatter(acca, [dn + (h * 10 + c)], e * xa)

    for i in range(G3):
        s = s3v[pl.ds(i * 16, 16)]
        d = d3v[pl.ds(i * 16, 16)]
        s128 = s * 128
        d128 = d * 128
        for h in range(H3):
            l = lbuf3[pl.ds(h * E3T + i * 16, 16)]
            e = jnp.exp(l - m3s[h])
            plsc.addupdate_scatter(acca, [d128 + (24 + h)], e)
            for c in range(C3):
                xa = plsc.load_gather(h3v, [s128 + (h * 16 + c)])
                plsc.addupdate_scatter(acca, [d128 + (B_N3 + h * 16 + c)],
                                       e * xa)

    pltpu.sync_copy(acca, slab.at[sid])
    plsc.subcore_barrier()

    @pl.when(sid == 0)
    def _():
        for slot in range(12):
            v = maxs[pl.ds(slot * 16, 16)]
            for j in range(8):
                maxrowv[pl.ds(slot * 128 + j * 16, 16)] = v
        pltpu.sync_copy(maxrowv, accp.at[cid, pl.ds(S_MAXR, 1536)])

    off = sid * CHUNK
    pulls = [pltpu.async_copy(slab.at[r, pl.ds(off, CHUNK)],
                              redv.at[r], sem)
             for r in range(16)]
    for c in pulls:
        c.wait()

    def chunk(j, c):
        v = redv[0, pl.ds(j * 16, 16)]
        for r in range(1, 16):
            v = v + redv[r, pl.ds(j * 16, 16)]
        resv[pl.ds(j * 16, 16)] = v
        return c

    lax.fori_loop(0, CHUNK // 16, chunk, 0)
    pltpu.sync_copy(resv, accp.at[cid, pl.ds(S_PK + off, CHUNK)])


_SC_OUT = jax.ShapeDtypeStruct((2, SZ_ACC), F32)

_SC_SCRATCH = [
    pltpu.VMEM((32,), F32),
    pltpu.VMEM((SZ_XW1,), F32),
    pltpu.VMEM((SZ_XW2,), F32),
    pltpu.VMEM((SZ_H3,), F32),
    pltpu.VMEM((SZ_AS,), F32),
    pltpu.VMEM((SZ_AD,), F32),
    pltpu.VMEM((E1T,), jnp.int32),
    pltpu.VMEM((E1T,), jnp.int32),
    pltpu.VMEM((E2T,), jnp.int32),
    pltpu.VMEM((E2T,), jnp.int32),
    pltpu.VMEM((E3T,), jnp.int32),
    pltpu.VMEM((E3T,), jnp.int32),
    pltpu.VMEM((SZ_A,), F32),
    pltpu.VMEM((H1 * E1T,), F32),
    pltpu.VMEM((E2T,), F32),
    pltpu.VMEM((H3 * E3T,), F32),
    pltpu.VMEM((SZ_MAX,), F32),
    pltpu.VMEM((16, SZ_MAX), F32),
    pltpu.VMEM((16, CHUNK), F32),
    pltpu.VMEM((CHUNK,), F32),
    pltpu.VMEM((1536,), F32),
    pltpu.VMEM_SHARED((16, SZ_MAX), F32),
    pltpu.VMEM_SHARED((16, SZ_A), F32),
    pltpu.SemaphoreType.DMA,
]


@functools.cache
def _sc_kernel():
    return functools.partial(
        pl.kernel,
        out_type=_SC_OUT,
        mesh=plsc.VectorSubcoreMesh(
            core_axis_name="c", subcore_axis_name="s",
            num_cores=2, num_subcores=16),
        scratch_types=_SC_SCRATCH,
        compiler_params=pltpu.CompilerParams(
            use_tc_tiling_on_sc=False, needs_layout_passes=False),
    )(_sc_body)


def _sc_call(*args):
    return _sc_kernel()(*args)


def _combine_body(acc, sel, e1, e3, wlin, b1, b2, blin, t1o, t2o, t3o):
    x = acc[...]
    mt = jnp.max(x[:, 0:12, :], axis=2)
    gm = jnp.max(mt, axis=0)
    sc = jnp.exp(mt - gm[None, :])
    e1m = e1[...]
    e3m = e3[...]

    spat1 = jnp.dot(sc, sel[...], preferred_element_type=F32)
    pk = jnp.sum(x[:, 12:116, :] * spat1[:, None, :], axis=0)
    num1 = pk[:, 0:20]
    den1 = pk[:, 20:22]
    den3 = pk[:, 24:32]
    dex1 = jnp.dot(den1, e1m, preferred_element_type=F32)
    t1o[...] = jnp.maximum(num1 / (dex1 + 1e-16) + b1[...], 0.0)

    spat3 = jnp.dot(sc[:, 3:11], e3m, preferred_element_type=F32)
    num3 = jnp.sum(x[:, 116:220, :] * spat3[:, None, :], axis=0)
    dex3 = jnp.dot(den3, e3m, preferred_element_type=F32)
    pre = jnp.maximum(num3 / (dex3 + 1e-16), 0.0)
    o3 = jnp.dot(pre, wlin[...], preferred_element_type=F32) + blin[...]
    t3o[...] = jnp.maximum(o3, 0.0)

    sc2 = sc[:, 2]
    den2 = jnp.sum(x[:, 220:239, :] * sc2[:, None, None], axis=0)
    num2 = jnp.sum(x[:, 239:258, :] * sc2[:, None, None], axis=0)
    t2o[...] = jnp.maximum(num2 / (den2 + 1e-16) + b2[...], 0.0)


def _final_body(x_ref, w_ref, bf_ref, o_ref):
    o_ref[...] = jnp.dot(x_ref[...], w_ref[...],
                         preferred_element_type=F32) + bf_ref[...]


def kernel(x1, edge_index1, x2, edge_index2, x3, edge_index3, Wl1, bl1, Wr1, br1, att1, bias1, Wl2, bl2, Wr2, br2, att2, bias2, Wp, bp, asrc, adst, Wk, bk, q, Wlin, blin, Wf, bf):
    i32 = edge_index1.dtype

    xw1, xw2, h3, as_, ad_, consts = pl.pallas_call(
        _proj_body,
        out_shape=[
            jax.ShapeDtypeStruct((N1, 40), F32),
            jax.ShapeDtypeStruct((N2, 2), F32),
            jax.ShapeDtypeStruct((N3, 128), F32),
            jax.ShapeDtypeStruct((N3, 8), F32),
            jax.ShapeDtypeStruct((N3, 8), F32),
            jax.ShapeDtypeStruct((1, 32), F32),
        ],
    )(x1, Wl1, bl1.reshape(1, 20), Wr1, br1.reshape(1, 20),
      x2, Wl2, bl2.reshape(1, 1), Wr2, br2.reshape(1, 1),
      x3, Wp, bp.reshape(1, 128), asrc, adst, att1, att2)

    lp1 = jnp.arange(N1, dtype=i32)
    zk = lambda k: jnp.zeros((k,), i32)
    dump = lambda n, k: (n + jnp.arange(k, dtype=i32) % 16).astype(i32)
    p1 = E1 - 1360 - N1
    p3 = E3 - 1360
    edges_small = jnp.concatenate([
        edge_index1[0], lp1, zk(p1),
        edge_index1[1], lp1, dump(N1, p1),
        edge_index3[0], zk(p3),
        edge_index3[1], dump(N3, p3),
    ])
    nodes = jnp.concatenate([
        xw1.reshape(N1 * 40), xw2.reshape(N2 * 2),
        as_.reshape(N3 * 8), ad_.reshape(N3 * 8),
        h3.reshape(SZ_H3), consts.reshape(32)])

    accp = _sc_call(nodes, edges_small, edge_index2)

    e1m = jnp.repeat(jnp.eye(2, dtype=F32), 10, axis=1)
    e3m = jnp.repeat(jnp.eye(8, dtype=F32), 16, axis=1)
    lane = jnp.arange(128)
    slot = jnp.arange(12)[:, None]
    sel = (((lane[None, :] < 20) & (slot == lane[None, :] // 10))
           | ((lane[None, :] >= 20) & (lane[None, :] < 22)
              & (slot == lane[None, :] - 20))
           | ((lane[None, :] >= 24) & (lane[None, :] < 32)
              & (slot == lane[None, :] - 24 + 3))).astype(F32)
    t1, t2, t3 = pl.pallas_call(
        _combine_body,
        out_shape=[
            jax.ShapeDtypeStruct((P, 20), F32),
            jax.ShapeDtypeStruct((19, 128), F32),
            jax.ShapeDtypeStruct((P, 20), F32),
        ],
    )(accp.reshape(2, 258, 128), sel, e1m, e3m, Wlin,
      bias1.reshape(1, 20), bias2.reshape(1, 1), blin.reshape(1, 20))

    xcat = jnp.concatenate(
        [t1[:N1].reshape(100, 17), t2.reshape(SZ_D2P)[:N2].reshape(100, 24),
         t3[:N3].reshape(100, 17)], axis=1)
    return pl.pallas_call(
        _final_body,
        out_shape=jax.ShapeDtypeStruct((100, 7), F32),
    )(xcat, Wf, bf.reshape(1, 7))


# speedup vs baseline: 18.4706x; 1.0032x over previous
import functools

import jax
import jax.numpy as jnp
from jax import lax
from jax.experimental import pallas as pl
from jax.experimental.pallas import tpu as pltpu
from jax.experimental.pallas import tpu_sc as plsc

F32 = jnp.float32
NEG = -3e38

N1, H1, C1 = 85, 2, 10
N2 = 2400
N3, H3, C3 = 85, 8, 16
P = 104
E1 = 1536
E2 = 40960
E3 = 1536
NW = 32
E1T, E2T, E3T = E1 // NW, E2 // NW, E3 // NW
G1, G2, G3 = E1T // 16, E2T // 16, E3T // 16

SZ_XW1 = 4160
SZ_XW2 = 4832
SZ_H3 = 10880
SZ_AS = 680
SZ_AD = 832
SZ_D1 = 208
SZ_N1 = 2080
SZ_D2 = 2416
SZ_D3 = 832
SZ_N3 = 13312
SZ_MAX = 192


def _proj_body(x1, wl1, bl1, wr1, br1, x2, wl2, bl2, wr2, br2,
               x3, wp, bp, asrc, adst, att1, att2,
               xw1, xw2, h3, as_, ad_, consts):
    h = jnp.dot(x3[...], wp[...], preferred_element_type=F32) + bp[...]
    h3[...] = h
    asm = asrc[...]
    adm = adst[...]
    acols, dcols = [], []
    for hh in range(8):
        blk = h[:, hh * 16:(hh + 1) * 16]
        acols.append(jnp.sum(blk * asm[hh:hh + 1, :], axis=1, keepdims=True))
        dcols.append(jnp.sum(blk * adm[hh:hh + 1, :], axis=1, keepdims=True))
    as_[...] = jnp.concatenate(acols, axis=1)
    ad_[...] = jnp.concatenate(dcols, axis=1)
    xw1[...] = jnp.concatenate(
        [jnp.dot(x1[...], wl1[...], preferred_element_type=F32) + bl1[...],
         jnp.dot(x1[...], wr1[...], preferred_element_type=F32) + br1[...]],
        axis=1)
    xw2[...] = jnp.concatenate(
        [jnp.dot(x2[...], wl2[...], preferred_element_type=F32) + bl2[...],
         jnp.dot(x2[...], wr2[...], preferred_element_type=F32) + br2[...]],
        axis=1)
    a1 = att1[...]
    consts[...] = jnp.concatenate(
        [a1[0:1, :], a1[1:2, :], att2[...], jnp.zeros((1, 11), F32)], axis=1)


def _zero(ref, n):
    z = jnp.zeros((16,), F32)

    def body(i, c):
        base = i * 128
        for j in range(8):
            ref[pl.ds(base + j * 16, 16)] = z
        return c
    lax.fori_loop(0, n // 128, body, 0)
    for k in range(n // 128 * 128, n, 16):
        ref[pl.ds(k, 16)] = z


O_XW1 = 0
O_XW2 = 3400
O_AS = 8200
O_AD = 8880
O_H3 = 9560
O_CONST = 20440
SZ_NODES = 20472

OFF_S1 = 0
OFF_D1 = E1
OFF_S3 = 2 * E1
OFF_D3 = 3 * E1

E2R = 38400 // NW
E2L = N2 // NW


S_MAXR = 0
S_PK = 1536
S_N3 = 14848
S_D2 = 28160
S_N2 = 30592
SZ_ACC = 33024
SZ_PK = 13312
SZ_D2P = 2432

B_PK = 0
B_N3 = 13312
B_D2 = 26624
B_N2 = 29056
SZ_A = 31488
CHUNK = SZ_A // 16


def _sc_body(nodes, edges_small, ei2, accp,
             constsv, xw1v, xw2v, h3v, asv, adv,
             s1v, d1v, s2v, d2v, s3v, d3v, acca,
             lbuf1, lbuf2, lbuf3, maxs, maxall, redv, resv, maxrowv,
             maxslab, slab, sem):
    cid = lax.axis_index("c")
    sid = lax.axis_index("s")
    wid = sid * 2 + cid
    z = jnp.zeros((16,), F32)

    es = edges_small
    copies = [
        pltpu.async_copy(nodes.at[pl.ds(O_CONST, 32)], constsv, sem),
        pltpu.async_copy(nodes.at[pl.ds(O_XW1, 3400)],
                         xw1v.at[pl.ds(0, 3400)], sem),
        pltpu.async_copy(nodes.at[pl.ds(O_XW2, 4800)],
                         xw2v.at[pl.ds(0, 4800)], sem),
        pltpu.async_copy(nodes.at[pl.ds(O_H3, SZ_H3)], h3v, sem),
        pltpu.async_copy(nodes.at[pl.ds(O_AS, SZ_AS)], asv, sem),
        pltpu.async_copy(nodes.at[pl.ds(O_AD, 680)],
                         adv.at[pl.ds(0, 680)], sem),
        pltpu.async_copy(es.at[pl.ds(OFF_S1 + wid * E1T, E1T)], s1v, sem),
        pltpu.async_copy(es.at[pl.ds(OFF_D1 + wid * E1T, E1T)], d1v, sem),
        pltpu.async_copy(ei2.at[0, pl.ds(wid * E2R, E2R)],
                         s2v.at[pl.ds(0, E2R)], sem),
        pltpu.async_copy(ei2.at[1, pl.ds(wid * E2R, E2R)],
                         d2v.at[pl.ds(0, E2R)], sem),
        pltpu.async_copy(es.at[pl.ds(OFF_S3 + wid * E3T, E3T)], s3v, sem),
        pltpu.async_copy(es.at[pl.ds(OFF_D3 + wid * E3T, E3T)], d3v, sem),
    ]

    iota = lax.iota(jnp.int32, 16)
    lbase = wid * E2L
    for k in range((E2T - E2R) // 16):
        idx = k * 16 + iota
        self_id = lbase + idx
        s2v[pl.ds(E2R + k * 16, 16)] = jnp.where(idx < E2L, self_id, 0)
        d2v[pl.ds(E2R + k * 16, 16)] = jnp.where(
            idx < E2L, self_id, N2 + idx - E2L)

    for k in list(range(3400, 4136, 16)) + [4144]:
        xw1v[pl.ds(k, 16)] = z
    xw2v[pl.ds(4800, 16)] = z
    xw2v[pl.ds(4816, 16)] = z
    for k in list(range(680, 808, 16)) + [816]:
        adv[pl.ds(k, 16)] = z
    _zero(acca, SZ_A)

    for c in copies:
        c.wait()

    neg = jnp.full((16,), NEG, F32)
    cv0 = constsv[pl.ds(0, 16)]
    cv1 = constsv[pl.ds(16, 16)]

    def att1c(j):
        return cv0[j] if j < 16 else cv1[j - 16]

    att2 = cv1[4]

    def g2p1(i, m):
        for u in range(2):
            g = i * 2 + u
            s = s2v[pl.ds(g * 16, 16)]
            d = d2v[pl.ds(g * 16, 16)]
            xa = plsc.load_gather(xw2v, [s * 2])
            xb = plsc.load_gather(xw2v, [d * 2 + 1])
            t = xa + xb
            l = att2 * jnp.where(t > 0, t, t * 0.2)
            lbuf2[pl.ds(g * 16, 16)] = l
            m = jnp.maximum(m, l)
        return m

    m2 = lax.fori_loop(0, G2 // 2, g2p1, neg)
    maxs[pl.ds(2 * 16, 16)] = m2

    m1 = [neg, neg]
    for i in range(G1):
        s = s1v[pl.ds(i * 16, 16)]
        d = d1v[pl.ds(i * 16, 16)]
        sb = s * 40
        db = d * 40
        for h in range(H1):
            acc = z
            for c in range(C1):
                xa = plsc.load_gather(xw1v, [sb + (h * 10 + c)])
                xb = plsc.load_gather(xw1v, [db + (20 + h * 10 + c)])
                t = xa + xb
                acc = acc + att1c(h * 10 + c) * jnp.where(t > 0, t, t * 0.2)
            lbuf1[pl.ds(h * E1T + i * 16, 16)] = acc
            m1[h] = jnp.maximum(m1[h], acc)
    for h in range(H1):
        maxs[pl.ds(h * 16, 16)] = m1[h]

    m3 = [neg] * H3
    for i in range(G3):
        s = s3v[pl.ds(i * 16, 16)]
        d = d3v[pl.ds(i * 16, 16)]
        s8 = s * 8
        d8 = d * 8
        for h in range(H3):
            xa = plsc.load_gather(asv, [s8 + h])
            xb = plsc.load_gather(adv, [d8 + h])
            t = xa + xb
            l = jnp.where(t > 0, t, t * 0.2)
            lbuf3[pl.ds(h * E3T + i * 16, 16)] = l
            m3[h] = jnp.maximum(m3[h], l)
    for h in range(H3):
        maxs[pl.ds((3 + h) * 16, 16)] = m3[h]
    maxs[pl.ds(11 * 16, 16)] = z

    pltpu.sync_copy(maxs, maxslab.at[sid])
    plsc.subcore_barrier()
    pltpu.sync_copy(maxslab, maxall)
    shift = []
    for slot in range(11):
        v = maxall[0, pl.ds(slot * 16, 16)]
        for r in range(1, 16):
            v = jnp.maximum(v, maxall[r, pl.ds(slot * 16, 16)])
        maxs[pl.ds(slot * 16, 16)] = v
        shift.append(jnp.max(v))
    m1s = shift[0:2]
    m2s = shift[2]
    m3s = shift[3:11]

    def g2p2(i, c):
        for u in range(2):
            g = i * 2 + u
            l = lbuf2[pl.ds(g * 16, 16)]
            e = jnp.exp(l - m2s)
            s = s2v[pl.ds(g * 16, 16)]
            d = d2v[pl.ds(g * 16, 16)]
            xa = plsc.load_gather(xw2v, [s * 2])
            plsc.addupdate_scatter(acca, [d + B_D2], e)
            plsc.addupdate_scatter(acca, [d + B_N2], e * xa)
        return c

    lax.fori_loop(0, G2 // 2, g2p2, 0)

    for i in range(G1):
        s = s1v[pl.ds(i * 16, 16)]
        d = d1v[pl.ds(i * 16, 16)]
        sb = s * 40
        dn = d * 128
        for h in range(H1):
            l = lbuf1[pl.ds(h * E1T + i * 16, 16)]
            e = jnp.exp(l - m1s[h])
            plsc.addupdate_scatter(acca, [dn + (20 + h)], e)
            for c in range(C1):
                xa = plsc.load_gather(xw1v, [sb + (h * 10 + c)])
                plsc.addupdate_scatter(acca, [dn + (h * 10 + c)], e * xa)

    for i in range(G3):
        s = s3v[pl.ds(i * 16, 16)]
        d = d3v[pl.ds(i * 16, 16)]
        s128 = s * 128
        d128 = d * 128
        for h in range(H3):
            l = lbuf3[pl.ds(h * E3T + i * 16, 16)]
            e = jnp.exp(l - m3s[h])
            plsc.addupdate_scatter(acca, [d128 + (24 + h)], e)
            for c in range(C3):
                xa = plsc.load_gather(h3v, [s128 + (h * 16 + c)])
                plsc.addupdate_scatter(acca, [d128 + (B_N3 + h * 16 + c)],
                                       e * xa)

    pltpu.sync_copy(acca, slab.at[sid])
    plsc.subcore_barrier()

    @pl.when(sid == 0)
    def _():
        for slot in range(12):
            v = maxs[pl.ds(slot * 16, 16)]
            for j in range(8):
                maxrowv[pl.ds(slot * 128 + j * 16, 16)] = v
        pltpu.sync_copy(maxrowv, accp.at[cid, pl.ds(S_MAXR, 1536)])

    off = sid * CHUNK
    pltpu.sync_copy(slab.at[:, pl.ds(off, CHUNK)], redv)

    def chunk(j, c):
        v = redv[0, pl.ds(j * 16, 16)]
        for r in range(1, 16):
            v = v + redv[r, pl.ds(j * 16, 16)]
        resv[pl.ds(j * 16, 16)] = v
        return c

    lax.fori_loop(0, CHUNK // 16, chunk, 0)
    pltpu.sync_copy(resv, accp.at[cid, pl.ds(S_PK + off, CHUNK)])


_SC_OUT = jax.ShapeDtypeStruct((2, SZ_ACC), F32)

_SC_SCRATCH = [
    pltpu.VMEM((32,), F32),
    pltpu.VMEM((SZ_XW1,), F32),
    pltpu.VMEM((SZ_XW2,), F32),
    pltpu.VMEM((SZ_H3,), F32),
    pltpu.VMEM((SZ_AS,), F32),
    pltpu.VMEM((SZ_AD,), F32),
    pltpu.VMEM((E1T,), jnp.int32),
    pltpu.VMEM((E1T,), jnp.int32),
    pltpu.VMEM((E2T,), jnp.int32),
    pltpu.VMEM((E2T,), jnp.int32),
    pltpu.VMEM((E3T,), jnp.int32),
    pltpu.VMEM((E3T,), jnp.int32),
    pltpu.VMEM((SZ_A,), F32),
    pltpu.VMEM((H1 * E1T,), F32),
    pltpu.VMEM((E2T,), F32),
    pltpu.VMEM((H3 * E3T,), F32),
    pltpu.VMEM((SZ_MAX,), F32),
    pltpu.VMEM((16, SZ_MAX), F32),
    pltpu.VMEM((16, CHUNK), F32),
    pltpu.VMEM((CHUNK,), F32),
    pltpu.VMEM((1536,), F32),
    pltpu.VMEM_SHARED((16, SZ_MAX), F32),
    pltpu.VMEM_SHARED((16, SZ_A), F32),
    pltpu.SemaphoreType.DMA,
]


@functools.cache
def _sc_kernel():
    return functools.partial(
        pl.kernel,
        out_type=_SC_OUT,
        mesh=plsc.VectorSubcoreMesh(
            core_axis_name="c", subcore_axis_name="s",
            num_cores=2, num_subcores=16),
        scratch_types=_SC_SCRATCH,
        compiler_params=pltpu.CompilerParams(
            use_tc_tiling_on_sc=False, needs_layout_passes=False),
    )(_sc_body)


def _sc_call(*args):
    return _sc_kernel()(*args)


def _combine_body(acc, sel, e1, e3, wlin, b1, b2, blin, t1o, t2o, t3o):
    x = acc[...]
    mt = jnp.max(x[:, 0:12, :], axis=2)
    gm = jnp.max(mt, axis=0)
    sc = jnp.exp(mt - gm[None, :])
    e1m = e1[...]
    e3m = e3[...]

    spat1 = jnp.dot(sc, sel[...], preferred_element_type=F32)
    pk = jnp.sum(x[:, 12:116, :] * spat1[:, None, :], axis=0)
    num1 = pk[:, 0:20]
    den1 = pk[:, 20:22]
    den3 = pk[:, 24:32]
    dex1 = jnp.dot(den1, e1m, preferred_element_type=F32)
    t1 = jnp.maximum(num1 / (dex1 + 1e-16) + b1[...], 0.0)
    t1o[...] = t1[0:N1, :]

    spat3 = jnp.dot(sc[:, 3:11], e3m, preferred_element_type=F32)
    num3 = jnp.sum(x[:, 116:220, :] * spat3[:, None, :], axis=0)
    dex3 = jnp.dot(den3, e3m, preferred_element_type=F32)
    pre = jnp.maximum(num3 / (dex3 + 1e-16), 0.0)
    o3 = jnp.dot(pre[0:N3, :], wlin[...], preferred_element_type=F32) + blin[...]
    t3o[...] = jnp.maximum(o3, 0.0)

    sc2 = sc[:, 2]
    den2 = jnp.sum(x[:, 220:239, :] * sc2[:, None, None], axis=0)
    num2 = jnp.sum(x[:, 239:258, :] * sc2[:, None, None], axis=0)
    t2o[...] = jnp.maximum(num2 / (den2 + 1e-16) + b2[...], 0.0)


def _final_body(x_ref, w_ref, bf_ref, o_ref):
    o_ref[...] = jnp.dot(x_ref[...], w_ref[...],
                         preferred_element_type=F32) + bf_ref[...]


def kernel(x1, edge_index1, x2, edge_index2, x3, edge_index3, Wl1, bl1, Wr1, br1, att1, bias1, Wl2, bl2, Wr2, br2, att2, bias2, Wp, bp, asrc, adst, Wk, bk, q, Wlin, blin, Wf, bf):
    i32 = edge_index1.dtype

    xw1, xw2, h3, as_, ad_, consts = pl.pallas_call(
        _proj_body,
        out_shape=[
            jax.ShapeDtypeStruct((N1, 40), F32),
            jax.ShapeDtypeStruct((N2, 2), F32),
            jax.ShapeDtypeStruct((N3, 128), F32),
            jax.ShapeDtypeStruct((N3, 8), F32),
            jax.ShapeDtypeStruct((N3, 8), F32),
            jax.ShapeDtypeStruct((1, 32), F32),
        ],
    )(x1, Wl1, bl1.reshape(1, 20), Wr1, br1.reshape(1, 20),
      x2, Wl2, bl2.reshape(1, 1), Wr2, br2.reshape(1, 1),
      x3, Wp, bp.reshape(1, 128), asrc, adst, att1, att2)

    lp1 = jnp.arange(N1, dtype=i32)
    zk = lambda k: jnp.zeros((k,), i32)
    dump = lambda n, k: (n + jnp.arange(k, dtype=i32) % 16).astype(i32)
    p1 = E1 - 1360 - N1
    p3 = E3 - 1360
    edges_small = jnp.concatenate([
        edge_index1[0], lp1, zk(p1),
        edge_index1[1], lp1, dump(N1, p1),
        edge_index3[0], zk(p3),
        edge_index3[1], dump(N3, p3),
    ])
    nodes = jnp.concatenate([
        xw1.reshape(N1 * 40), xw2.reshape(N2 * 2),
        as_.reshape(N3 * 8), ad_.reshape(N3 * 8),
        h3.reshape(SZ_H3), consts.reshape(32)])

    accp = _sc_call(nodes, edges_small, edge_index2)

    e1m = jnp.repeat(jnp.eye(2, dtype=F32), 10, axis=1)
    e3m = jnp.repeat(jnp.eye(8, dtype=F32), 16, axis=1)
    lane = jnp.arange(128)
    slot = jnp.arange(12)[:, None]
    sel = (((lane[None, :] < 20) & (slot == lane[None, :] // 10))
           | ((lane[None, :] >= 20) & (lane[None, :] < 22)
              & (slot == lane[None, :] - 20))
           | ((lane[None, :] >= 24) & (lane[None, :] < 32)
              & (slot == lane[None, :] - 24 + 3))).astype(F32)
    t1, t2, t3 = pl.pallas_call(
        _combine_body,
        out_shape=[
            jax.ShapeDtypeStruct((N1, 20), F32),
            jax.ShapeDtypeStruct((19, 128), F32),
            jax.ShapeDtypeStruct((N3, 20), F32),
        ],
    )(accp.reshape(2, 258, 128), sel, e1m, e3m, Wlin,
      bias1.reshape(1, 20), bias2.reshape(1, 1), blin.reshape(1, 20))

    xcat = jnp.concatenate(
        [t1.reshape(100, 17), t2.reshape(SZ_D2P)[:N2].reshape(100, 24),
         t3.reshape(100, 17)], axis=1)
    return pl.pallas_call(
        _final_body,
        out_shape=jax.ShapeDtypeStruct((100, 7), F32),
    )(xcat, Wf, bf.reshape(1, 7))
